# Initial kernel scaffold; baseline (speedup 1.0000x reference)
import jax
import jax.numpy as jnp
from jax.experimental import pallas as pl


def kernel(x, edge_index, batch, W1, b1, W2, b2, W3, b3):
    raise NotImplementedError("write your pallas kernel here")


# speedup vs baseline: 14.1808x; 14.1808x over previous
import functools

import jax
import jax.numpy as jnp
from jax import lax
from jax.experimental import pallas as pl
from jax.experimental.pallas import tpu as pltpu
from jax.experimental.pallas import tpu_sc as plsc

N = 100000
E = 1600000
G = 32
NP = 100352
STRIPE = NP // 16
ZROWS = STRIPE // 8
NCORES = 2
EPT = 51200
NCHUNK = 25
EPAD = NCORES * 16 * EPT


def _sc_mesh():
    return plsc.VectorSubcoreMesh(core_axis_name="c", subcore_axis_name="s")


@functools.partial(
    pl.kernel,
    out_type=jax.ShapeDtypeStruct((NCORES, NP), jnp.float32),
    mesh=_sc_mesh(),
    compiler_params=pltpu.CompilerParams(use_tc_tiling_on_sc=False),
    scratch_types=[
        pltpu.VMEM((16, 128), jnp.int32),
        pltpu.VMEM((128,), jnp.float32),
        pltpu.VMEM((ZROWS,), jnp.float32),
        pltpu.VMEM_SHARED((NP,), jnp.float32),
    ],
)
def _deg_kernel(dst_hbm, out_hbm, dstv, ones_v, zb, acc):
    c = lax.axis_index("c")
    s = lax.axis_index("s")

    for i in range(8):
        ones_v[pl.ds(16 * i, 16)] = jnp.ones((16,), jnp.float32)

    def zinit(i, carry):
        zb[pl.ds(16 * i, 16)] = jnp.zeros((16,), jnp.float32)
        return carry

    lax.fori_loop(0, ZROWS // 16, zinit, 0)
    for k in range(8):
        pltpu.sync_copy(zb, acc.at[pl.ds(s * STRIPE + k * ZROWS, ZROWS)])
    plsc.subcore_barrier()

    def chunk(ci, carry):
        pltpu.sync_copy(dst_hbm.at[c, s, ci], dstv)

        def inner(j, carry2):
            pltpu.sync_copy(ones_v, acc.at[dstv.at[j]], add=True)
            return carry2

        return lax.fori_loop(0, 16, inner, carry)

    lax.fori_loop(0, NCHUNK, chunk, 0)
    plsc.subcore_barrier()
    pltpu.sync_copy(acc.at[pl.ds(s * STRIPE, STRIPE)],
                    out_hbm.at[c, pl.ds(s * STRIPE, STRIPE)])


def _make_prop(nsl):
    @functools.partial(
        pl.kernel,
        out_type=jax.ShapeDtypeStruct((NCORES, NP, 16 * nsl), jnp.float32),
        mesh=_sc_mesh(),
        compiler_params=pltpu.CompilerParams(use_tc_tiling_on_sc=False),
        scratch_types=[
            pltpu.VMEM((16, 128), jnp.int32),
            pltpu.VMEM((16, 128), jnp.int32),
            pltpu.VMEM((128, 16), jnp.float32),
            pltpu.VMEM((ZROWS, 16), jnp.float32),
            pltpu.VMEM_SHARED((NP, 16), jnp.float32),
            pltpu.SemaphoreType.DMA,
        ],
    )
    def _prop(*refs):
        gs = refs[0:nsl]
        src_hbm, dst_hbm, out_hbm = refs[nsl], refs[nsl + 1], refs[nsl + 2]
        srcv, dstv, rows, zb, acc, sem = refs[nsl + 3:]
        c = lax.axis_index("c")
        s = lax.axis_index("s")

        def zinit(i, carry):
            zb[i, :] = jnp.zeros((16,), jnp.float32)
            return carry

        lax.fori_loop(0, ZROWS, zinit, 0)

        for sl in range(nsl):
            for k in range(8):
                pltpu.sync_copy(
                    zb, acc.at[pl.ds(s * STRIPE + k * ZROWS, ZROWS)])
            plsc.subcore_barrier()

            def chunk(ci, carry):
                pltpu.sync_copy(src_hbm.at[c, s, ci], srcv)
                pltpu.sync_copy(dst_hbm.at[c, s, ci], dstv)

                def inner(j, carry2):
                    pltpu.async_copy(gs[sl].at[srcv.at[j]], rows, sem).wait()
                    pltpu.sync_copy(rows, acc.at[dstv.at[j]], add=True)
                    return carry2

                return lax.fori_loop(0, 16, inner, carry)

            lax.fori_loop(0, NCHUNK, chunk, 0)
            plsc.subcore_barrier()
            pltpu.sync_copy(
                acc.at[pl.ds(s * STRIPE, STRIPE)],
                out_hbm.at[c, pl.ds(s * STRIPE, STRIPE), pl.ds(16 * sl, 16)])

    return _prop


_prop1 = _make_prop(1)
_prop4 = _make_prop(4)


_TC_GRID = 16
_RB = STRIPE


def _row_spec(cols):
    return pl.BlockSpec((_RB, cols), lambda i: (i, 0))


def _pair_spec(cols):
    return pl.BlockSpec((NCORES, _RB, cols), lambda i: (0, i, 0))


def _full_spec(r, c):
    return pl.BlockSpec((r, c), lambda i: (0, 0))


def _tc1_body(deg_ref, xp_ref, dinv_ref, g1_ref):
    deg = deg_ref[0, :] + deg_ref[1, :] + 1.0
    dinv = lax.rsqrt(deg)[:, None]
    dinv16 = jnp.broadcast_to(dinv, (_RB, 16))
    dinv_ref[...] = dinv16
    g1_ref[...] = dinv16 * xp_ref[...]


def _tc1(deg2, xp):
    return pl.pallas_call(
        _tc1_body,
        grid=(_TC_GRID,),
        compiler_params=pltpu.CompilerParams(vmem_limit_bytes=100 * 2**20),
        in_specs=[pl.BlockSpec((NCORES, _RB), lambda i: (0, i)),
                  _row_spec(16)],
        out_specs=[_row_spec(16), _row_spec(16)],
        out_shape=[jax.ShapeDtypeStruct((NP, 16), jnp.float32),
                   jax.ShapeDtypeStruct((NP, 16), jnp.float32)],
    )(deg2, xp)


def _tc2_body(acc1_ref, g1_ref, dinv_ref, w1_ref, b1_ref, *g2_refs):
    dinv = dinv_ref[...]
    p1 = dinv * (acc1_ref[0] + acc1_ref[1] + g1_ref[...])
    h1 = jnp.maximum(
        jnp.dot(p1, w1_ref[...], preferred_element_type=jnp.float32,
                precision=lax.Precision.HIGHEST) + b1_ref[...], 0.0)
    for sl in range(4):
        g2_refs[sl][...] = dinv * h1[:, 16 * sl:16 * sl + 16]


def _tc2(acc1, g1, dinv16, w1p, b1):
    return pl.pallas_call(
        _tc2_body,
        grid=(_TC_GRID,),
        compiler_params=pltpu.CompilerParams(vmem_limit_bytes=100 * 2**20),
        in_specs=[_pair_spec(16), _row_spec(16), _row_spec(16),
                  _full_spec(16, 64), _full_spec(1, 64)],
        out_specs=[_row_spec(16)] * 4,
        out_shape=[jax.ShapeDtypeStruct((NP, 16), jnp.float32)] * 4,
    )(acc1, g1, dinv16, w1p, b1)


def _tc3_body(acc2_ref, g2a_ref, g2b_ref, g2c_ref, g2d_ref, dinv_ref,
              w2_ref, b2_ref, w3_ref, g3_ref):
    dinv = dinv_ref[...]
    dinv64 = jnp.concatenate([dinv, dinv, dinv, dinv], axis=1)
    g2 = jnp.concatenate(
        [g2a_ref[...], g2b_ref[...], g2c_ref[...], g2d_ref[...]], axis=1)
    p2 = dinv64 * (acc2_ref[0] + acc2_ref[1] + g2)
    h2 = jnp.maximum(
        jnp.dot(p2, w2_ref[...], preferred_element_type=jnp.float32,
                precision=lax.Precision.HIGHEST) + b2_ref[...], 0.0)
    t3 = jnp.dot(h2, w3_ref[...], preferred_element_type=jnp.float32,
                 precision=lax.Precision.HIGHEST)
    g3_ref[...] = dinv * t3


def _tc3(acc2, g2s, dinv16, w2, b2, w3p):
    return pl.pallas_call(
        _tc3_body,
        grid=(_TC_GRID,),
        compiler_params=pltpu.CompilerParams(vmem_limit_bytes=100 * 2**20),
        in_specs=[_pair_spec(64)] + [_row_spec(16)] * 4 +
                 [_row_spec(16), _full_spec(64, 64), _full_spec(1, 64),
                  _full_spec(64, 16)],
        out_specs=[_row_spec(16)],
        out_shape=[jax.ShapeDtypeStruct((NP, 16), jnp.float32)],
    )(acc2, *g2s, dinv16, w2, b2, w3p)


def _tc4_body(acc3_ref, g3_ref, dinv_ref, b3_ref, batch_ref, out_ref,
              sums_ref, cnts_ref):
    i = pl.program_id(0)

    @pl.when(i == 0)
    def _():
        sums_ref[...] = jnp.zeros((G, 16), jnp.float32)
        cnts_ref[...] = jnp.zeros((G, 16), jnp.float32)

    o = dinv_ref[...] * (acc3_ref[0] + acc3_ref[1] + g3_ref[...]) + b3_ref[...]
    b = batch_ref[...][:, 0:1]
    onehot = (b == lax.broadcasted_iota(jnp.int32, (_RB, G), 1)
              ).astype(jnp.float32)
    sums_ref[...] += lax.dot_general(
        onehot, o, (((0,), (0,)), ((), ())),
        preferred_element_type=jnp.float32,
        precision=lax.Precision.HIGHEST)
    cnts_ref[...] += lax.dot_general(
        onehot, jnp.ones((_RB, 16), jnp.float32), (((0,), (0,)), ((), ())),
        preferred_element_type=jnp.float32,
        precision=lax.Precision.HIGHEST)

    @pl.when(i == _TC_GRID - 1)
    def _():
        out_ref[...] = sums_ref[...] / jnp.maximum(cnts_ref[...], 1.0)


def _tc4(acc3, g3, dinv16, b3p, batch2d):
    return pl.pallas_call(
        _tc4_body,
        grid=(_TC_GRID,),
        compiler_params=pltpu.CompilerParams(vmem_limit_bytes=100 * 2**20),
        in_specs=[_pair_spec(16), _row_spec(16), _row_spec(16),
                  _full_spec(1, 16), _row_spec(16)],
        out_specs=[_full_spec(G, 16)],
        out_shape=[jax.ShapeDtypeStruct((G, 16), jnp.float32)],
        scratch_shapes=[pltpu.VMEM((G, 16), jnp.float32),
                        pltpu.VMEM((G, 16), jnp.float32)],
    )(acc3, g3, dinv16, b3p, batch2d)


@jax.jit
def _run(x, edge_index, batch, W1, b1, W2, b2, W3, b3):
    src = edge_index[0].astype(jnp.int32)
    dst = edge_index[1].astype(jnp.int32)
    pad = N + (jnp.arange(EPAD - E, dtype=jnp.int32) % (NP - N))
    srcp = jnp.concatenate([src, pad]).reshape(NCORES, 16, NCHUNK, 16, 128)
    dstp = jnp.concatenate([dst, pad]).reshape(NCORES, 16, NCHUNK, 16, 128)

    xp = jnp.pad(x, ((0, NP - N), (0, 13)))
    w1p = jnp.pad(W1, ((0, 13), (0, 0)))
    w3p = jnp.pad(W3, ((0, 0), (0, 11)))
    b1r = b1.reshape(1, 64)
    b2r = b2.reshape(1, 64)
    b3p = jnp.pad(b3, (0, 11)).reshape(1, 16)
    batchp = jnp.concatenate(
        [batch.astype(jnp.int32),
         jnp.full((NP - N,), 99, jnp.int32)])
    batch2d = jnp.broadcast_to(batchp[:, None], (NP, 16))

    deg2 = _deg_kernel(dstp)
    dinv16, g1 = _tc1(deg2, xp)
    acc1 = _prop1(g1, srcp, dstp)
    g2s = _tc2(acc1, g1, dinv16, w1p, b1r)
    acc2 = _prop4(*g2s, srcp, dstp)
    (g3,) = _tc3(acc2, g2s, dinv16, W2, b2r, w3p)
    acc3 = _prop1(g3, srcp, dstp)
    (pooled,) = _tc4(acc3, g3, dinv16, b3p, batch2d)
    return pooled[:, :5]


def kernel(x, edge_index, batch, W1, b1, W2, b2, W3, b3):
    return _run(x, edge_index, batch, W1, b1, W2, b2, W3, b3)


# speedup vs baseline: 23.1269x; 1.6309x over previous
import functools

import jax
import jax.numpy as jnp
from jax import lax
from jax.experimental import pallas as pl
from jax.experimental.pallas import tpu as pltpu
from jax.experimental.pallas import tpu_sc as plsc

N = 100000
E = 1600000
G = 32
NP = 100352
STRIPE = NP // 16
ZROWS = STRIPE // 8
NCORES = 2
NJ = 5
NCHUNK = 80
EPT = NCHUNK * NJ * 128
EPAD = NCORES * 16 * EPT


def _sc_mesh():
    return plsc.VectorSubcoreMesh(core_axis_name="c", subcore_axis_name="s")


@functools.partial(
    pl.kernel,
    out_type=jax.ShapeDtypeStruct((NCORES, NP), jnp.float32),
    mesh=_sc_mesh(),
    compiler_params=pltpu.CompilerParams(use_tc_tiling_on_sc=False),
    scratch_types=[
        pltpu.VMEM((NJ, 128), jnp.int32),
        pltpu.VMEM((128,), jnp.float32),
        pltpu.VMEM((ZROWS,), jnp.float32),
        pltpu.VMEM_SHARED((NP,), jnp.float32),
    ],
)
def _deg_kernel(dst_hbm, out_hbm, dstv, ones_v, zb, acc):
    c = lax.axis_index("c")
    s = lax.axis_index("s")

    for i in range(8):
        ones_v[pl.ds(16 * i, 16)] = jnp.ones((16,), jnp.float32)

    def zinit(i, carry):
        zb[pl.ds(16 * i, 16)] = jnp.zeros((16,), jnp.float32)
        return carry

    lax.fori_loop(0, ZROWS // 16, zinit, 0)
    for k in range(8):
        pltpu.sync_copy(zb, acc.at[pl.ds(s * STRIPE + k * ZROWS, ZROWS)])
    plsc.subcore_barrier()

    def chunk(ci, carry):
        pltpu.sync_copy(dst_hbm.at[c, s, ci], dstv)

        def inner(j, carry2):
            pltpu.sync_copy(ones_v, acc.at[dstv.at[j]], add=True)
            return carry2

        return lax.fori_loop(0, NJ, inner, carry)

    lax.fori_loop(0, NCHUNK, chunk, 0)
    plsc.subcore_barrier()
    pltpu.sync_copy(acc.at[pl.ds(s * STRIPE, STRIPE)],
                    out_hbm.at[c, pl.ds(s * STRIPE, STRIPE)])


def _make_prop(nsl):
    @functools.partial(
        pl.kernel,
        out_type=jax.ShapeDtypeStruct((NCORES, NP, 16 * nsl), jnp.float32),
        mesh=_sc_mesh(),
        compiler_params=pltpu.CompilerParams(use_tc_tiling_on_sc=False),
        scratch_types=[
            pltpu.VMEM((2, NJ, 128), jnp.int32),
            pltpu.VMEM((2, NJ, 128), jnp.int32),
            pltpu.VMEM((2, NJ * 128, 16), jnp.float32),
            pltpu.VMEM_SHARED((NP, 16), jnp.float32),
            pltpu.SemaphoreType.DMA,
            pltpu.SemaphoreType.DMA,
        ],
    )
    def _prop(*refs):
        gs = refs[0:nsl]
        src_hbm, dst_hbm = refs[nsl], refs[nsl + 1]
        zeros_hbm, out_hbm = refs[nsl + 2], refs[nsl + 3]
        srcv, dstv, rows, acc = refs[nsl + 4:nsl + 8]
        semg = refs[nsl + 8:nsl + 10]
        c = lax.axis_index("c")
        s = lax.axis_index("s")

        for sl in range(nsl):
            g = gs[sl]

            def fire_g(p, ci):
                pltpu.sync_copy(src_hbm.at[c, s, ci], srcv.at[p])
                pltpu.sync_copy(dst_hbm.at[c, s, ci], dstv.at[p])
                for j in range(NJ):
                    pltpu.async_copy(
                        g.at[srcv.at[p, j]],
                        rows.at[p, pl.ds(128 * j, 128), :], semg[p])

            def drain_g(p):
                for j in range(NJ):
                    pltpu.make_async_copy(
                        g.at[srcv.at[p, j]],
                        rows.at[p, pl.ds(128 * j, 128), :], semg[p]).wait()

            def scat(p):
                for j in range(NJ):
                    pltpu.sync_copy(
                        rows.at[p, pl.ds(128 * j, 128), :],
                        acc.at[dstv.at[p, j]], add=True)

            pltpu.sync_copy(zeros_hbm, acc.at[pl.ds(s * STRIPE, STRIPE)])
            plsc.subcore_barrier()

            fire_g(0, 0)

            def pair(i, carry):
                fire_g(1, 2 * i + 1)
                drain_g(0)
                scat(0)
                fire_g(0, 2 * i + 2)
                drain_g(1)
                scat(1)
                return carry

            lax.fori_loop(0, NCHUNK // 2 - 1, pair, 0)
            fire_g(1, NCHUNK - 1)
            drain_g(0)
            scat(0)
            drain_g(1)
            scat(1)

            plsc.subcore_barrier()
            pltpu.sync_copy(
                acc.at[pl.ds(s * STRIPE, STRIPE)],
                out_hbm.at[c, pl.ds(s * STRIPE, STRIPE), pl.ds(16 * sl, 16)])

    return _prop


_prop1 = _make_prop(1)
_prop4 = _make_prop(4)


_TC_GRID = 16
_RB = STRIPE


def _row_spec(cols):
    return pl.BlockSpec((_RB, cols), lambda i: (i, 0))


def _pair_spec(cols):
    return pl.BlockSpec((NCORES, _RB, cols), lambda i: (0, i, 0))


def _full_spec(r, c):
    return pl.BlockSpec((r, c), lambda i: (0, 0))


def _tc1_body(deg_ref, xp_ref, dinv_ref, g1_ref):
    deg = deg_ref[0, :] + deg_ref[1, :] + 1.0
    dinv = lax.rsqrt(deg)[:, None]
    dinv16 = jnp.broadcast_to(dinv, (_RB, 16))
    dinv_ref[...] = dinv16
    g1_ref[...] = dinv16 * xp_ref[...]


def _tc1(deg2, xp):
    return pl.pallas_call(
        _tc1_body,
        grid=(_TC_GRID,),
        compiler_params=pltpu.CompilerParams(vmem_limit_bytes=100 * 2**20),
        in_specs=[pl.BlockSpec((NCORES, _RB), lambda i: (0, i)),
                  _row_spec(16)],
        out_specs=[_row_spec(16), _row_spec(16)],
        out_shape=[jax.ShapeDtypeStruct((NP, 16), jnp.float32),
                   jax.ShapeDtypeStruct((NP, 16), jnp.float32)],
    )(deg2, xp)


def _tc2_body(acc1_ref, g1_ref, dinv_ref, w1_ref, b1_ref, *g2_refs):
    dinv = dinv_ref[...]
    p1 = dinv * (acc1_ref[0] + acc1_ref[1] + g1_ref[...])
    h1 = jnp.maximum(
        jnp.dot(p1, w1_ref[...], preferred_element_type=jnp.float32,
                precision=lax.Precision.HIGHEST) + b1_ref[...], 0.0)
    for sl in range(4):
        g2_refs[sl][...] = dinv * h1[:, 16 * sl:16 * sl + 16]


def _tc2(acc1, g1, dinv16, w1p, b1):
    return pl.pallas_call(
        _tc2_body,
        grid=(_TC_GRID,),
        compiler_params=pltpu.CompilerParams(vmem_limit_bytes=100 * 2**20),
        in_specs=[_pair_spec(16), _row_spec(16), _row_spec(16),
                  _full_spec(16, 64), _full_spec(1, 64)],
        out_specs=[_row_spec(16)] * 4,
        out_shape=[jax.ShapeDtypeStruct((NP, 16), jnp.float32)] * 4,
    )(acc1, g1, dinv16, w1p, b1)


def _tc3_body(acc2_ref, g2a_ref, g2b_ref, g2c_ref, g2d_ref, dinv_ref,
              w2_ref, b2_ref, w3_ref, g3_ref):
    dinv = dinv_ref[...]
    dinv64 = jnp.concatenate([dinv, dinv, dinv, dinv], axis=1)
    g2 = jnp.concatenate(
        [g2a_ref[...], g2b_ref[...], g2c_ref[...], g2d_ref[...]], axis=1)
    p2 = dinv64 * (acc2_ref[0] + acc2_ref[1] + g2)
    h2 = jnp.maximum(
        jnp.dot(p2, w2_ref[...], preferred_element_type=jnp.float32,
                precision=lax.Precision.HIGHEST) + b2_ref[...], 0.0)
    t3 = jnp.dot(h2, w3_ref[...], preferred_element_type=jnp.float32,
                 precision=lax.Precision.HIGHEST)
    g3_ref[...] = dinv * t3


def _tc3(acc2, g2s, dinv16, w2, b2, w3p):
    return pl.pallas_call(
        _tc3_body,
        grid=(_TC_GRID,),
        compiler_params=pltpu.CompilerParams(vmem_limit_bytes=100 * 2**20),
        in_specs=[_pair_spec(64)] + [_row_spec(16)] * 4 +
                 [_row_spec(16), _full_spec(64, 64), _full_spec(1, 64),
                  _full_spec(64, 16)],
        out_specs=[_row_spec(16)],
        out_shape=[jax.ShapeDtypeStruct((NP, 16), jnp.float32)],
    )(acc2, *g2s, dinv16, w2, b2, w3p)


def _tc4_body(acc3_ref, g3_ref, dinv_ref, b3_ref, batch_ref, out_ref,
              sums_ref, cnts_ref):
    i = pl.program_id(0)

    @pl.when(i == 0)
    def _():
        sums_ref[...] = jnp.zeros((G, 16), jnp.float32)
        cnts_ref[...] = jnp.zeros((G, 16), jnp.float32)

    o = dinv_ref[...] * (acc3_ref[0] + acc3_ref[1] + g3_ref[...]) + b3_ref[...]
    b = batch_ref[...][:, 0:1]
    onehot = (b == lax.broadcasted_iota(jnp.int32, (_RB, G), 1)
              ).astype(jnp.float32)
    sums_ref[...] += lax.dot_general(
        onehot, o, (((0,), (0,)), ((), ())),
        preferred_element_type=jnp.float32,
        precision=lax.Precision.HIGHEST)
    cnts_ref[...] += lax.dot_general(
        onehot, jnp.ones((_RB, 16), jnp.float32), (((0,), (0,)), ((), ())),
        preferred_element_type=jnp.float32,
        precision=lax.Precision.HIGHEST)

    @pl.when(i == _TC_GRID - 1)
    def _():
        out_ref[...] = sums_ref[...] / jnp.maximum(cnts_ref[...], 1.0)


def _tc4(acc3, g3, dinv16, b3p, batch2d):
    return pl.pallas_call(
        _tc4_body,
        grid=(_TC_GRID,),
        compiler_params=pltpu.CompilerParams(vmem_limit_bytes=100 * 2**20),
        in_specs=[_pair_spec(16), _row_spec(16), _row_spec(16),
                  _full_spec(1, 16), _row_spec(16)],
        out_specs=[_full_spec(G, 16)],
        out_shape=[jax.ShapeDtypeStruct((G, 16), jnp.float32)],
        scratch_shapes=[pltpu.VMEM((G, 16), jnp.float32),
                        pltpu.VMEM((G, 16), jnp.float32)],
    )(acc3, g3, dinv16, b3p, batch2d)


@jax.jit
def _run(x, edge_index, batch, W1, b1, W2, b2, W3, b3):
    src = edge_index[0].astype(jnp.int32)
    dst = edge_index[1].astype(jnp.int32)
    pad = N + (jnp.arange(EPAD - E, dtype=jnp.int32) % (NP - N))
    srcp = jnp.concatenate([src, pad]).reshape(NCORES, 16, NCHUNK, NJ, 128)
    dstp = jnp.concatenate([dst, pad]).reshape(NCORES, 16, NCHUNK, NJ, 128)

    xp = jnp.pad(x, ((0, NP - N), (0, 13)))
    w1p = jnp.pad(W1, ((0, 13), (0, 0)))
    w3p = jnp.pad(W3, ((0, 0), (0, 11)))
    b1r = b1.reshape(1, 64)
    b2r = b2.reshape(1, 64)
    b3p = jnp.pad(b3, (0, 11)).reshape(1, 16)
    batchp = jnp.concatenate(
        [batch.astype(jnp.int32),
         jnp.full((NP - N,), 99, jnp.int32)])
    batch2d = jnp.broadcast_to(batchp[:, None], (NP, 16))

    zrs = jnp.zeros((STRIPE, 16), jnp.float32)
    deg2 = _deg_kernel(dstp)
    dinv16, g1 = _tc1(deg2, xp)
    acc1 = _prop1(g1, srcp, dstp, zrs)
    g2s = _tc2(acc1, g1, dinv16, w1p, b1r)
    acc2 = _prop4(*g2s, srcp, dstp, zrs)
    (g3,) = _tc3(acc2, g2s, dinv16, W2, b2r, w3p)
    acc3 = _prop1(g3, srcp, dstp, zrs)
    (pooled,) = _tc4(acc3, g3, dinv16, b3p, batch2d)
    return pooled[:, :5]


def kernel(x, edge_index, batch, W1, b1, W2, b2, W3, b3):
    return _run(x, edge_index, batch, W1, b1, W2, b2, W3, b3)


# speedup vs baseline: 32.3178x; 1.3974x over previous
import functools

import jax
import jax.numpy as jnp
from jax import lax
from jax.experimental import pallas as pl
from jax.experimental.pallas import tpu as pltpu
from jax.experimental.pallas import tpu_sc as plsc

N = 100000
E = 1600000
G = 32
NP = 100352
STRIPE = NP // 16
ZROWS = STRIPE // 8
NCORES = 2
NJ = 5
NCHUNK = 80
EPT = NCHUNK * NJ * 128
EPAD = NCORES * 16 * EPT


def _sc_mesh():
    return plsc.VectorSubcoreMesh(core_axis_name="c", subcore_axis_name="s")


@functools.partial(
    pl.kernel,
    out_type=jax.ShapeDtypeStruct((NCORES, NP), jnp.float32),
    mesh=_sc_mesh(),
    compiler_params=pltpu.CompilerParams(use_tc_tiling_on_sc=False),
    scratch_types=[
        pltpu.VMEM((NJ, 128), jnp.int32),
        pltpu.VMEM((128,), jnp.float32),
        pltpu.VMEM((ZROWS,), jnp.float32),
        pltpu.VMEM_SHARED((NP,), jnp.float32),
    ],
)
def _deg_kernel(dst_hbm, out_hbm, dstv, ones_v, zb, acc):
    c = lax.axis_index("c")
    s = lax.axis_index("s")

    for i in range(8):
        ones_v[pl.ds(16 * i, 16)] = jnp.ones((16,), jnp.float32)

    def zinit(i, carry):
        zb[pl.ds(16 * i, 16)] = jnp.zeros((16,), jnp.float32)
        return carry

    lax.fori_loop(0, ZROWS // 16, zinit, 0)
    for k in range(8):
        pltpu.sync_copy(zb, acc.at[pl.ds(s * STRIPE + k * ZROWS, ZROWS)])
    plsc.subcore_barrier()

    def chunk(ci, carry):
        pltpu.sync_copy(dst_hbm.at[c, s, ci], dstv)

        def inner(j, carry2):
            pltpu.sync_copy(ones_v, acc.at[dstv.at[j]], add=True)
            return carry2

        return lax.fori_loop(0, NJ, inner, carry)

    lax.fori_loop(0, NCHUNK, chunk, 0)
    plsc.subcore_barrier()
    pltpu.sync_copy(acc.at[pl.ds(s * STRIPE, STRIPE)],
                    out_hbm.at[c, pl.ds(s * STRIPE, STRIPE)])


def _make_prop(nsl):
    @functools.partial(
        pl.kernel,
        out_type=jax.ShapeDtypeStruct((NCORES, nsl, NP, 16), jnp.float32),
        mesh=_sc_mesh(),
        compiler_params=pltpu.CompilerParams(use_tc_tiling_on_sc=False),
        scratch_types=[
            pltpu.VMEM((2, NJ, 128), jnp.int32),
            pltpu.VMEM((2, NJ, 128), jnp.int32),
            pltpu.VMEM((2, NJ * 128, 16), jnp.float32),
            pltpu.VMEM_SHARED((NP, 16), jnp.float32),
            pltpu.SemaphoreType.DMA,
            pltpu.SemaphoreType.DMA,
        ],
    )
    def _prop(*refs):
        gs = refs[0:nsl]
        src_hbm, dst_hbm = refs[nsl], refs[nsl + 1]
        zeros_hbm, out_hbm = refs[nsl + 2], refs[nsl + 3]
        srcv, dstv, rows, acc = refs[nsl + 4:nsl + 8]
        semg = refs[nsl + 8:nsl + 10]
        c = lax.axis_index("c")
        s = lax.axis_index("s")

        for sl in range(nsl):
            g = gs[sl]

            def fire_g(p, ci):
                pltpu.sync_copy(src_hbm.at[c, s, ci], srcv.at[p])
                pltpu.sync_copy(dst_hbm.at[c, s, ci], dstv.at[p])
                for j in range(NJ):
                    pltpu.async_copy(
                        g.at[srcv.at[p, j]],
                        rows.at[p, pl.ds(128 * j, 128), :], semg[p])

            def drain_g(p):
                for j in range(NJ):
                    pltpu.make_async_copy(
                        g.at[srcv.at[p, j]],
                        rows.at[p, pl.ds(128 * j, 128), :], semg[p]).wait()

            def scat(p):
                for j in range(NJ):
                    pltpu.sync_copy(
                        rows.at[p, pl.ds(128 * j, 128), :],
                        acc.at[dstv.at[p, j]], add=True)

            pltpu.sync_copy(zeros_hbm, acc.at[pl.ds(s * STRIPE, STRIPE)])
            plsc.subcore_barrier()

            fire_g(0, 0)

            def pair(i, carry):
                fire_g(1, 2 * i + 1)
                drain_g(0)
                scat(0)
                fire_g(0, 2 * i + 2)
                drain_g(1)
                scat(1)
                return carry

            lax.fori_loop(0, NCHUNK // 2 - 1, pair, 0)
            fire_g(1, NCHUNK - 1)
            drain_g(0)
            scat(0)
            drain_g(1)
            scat(1)

            plsc.subcore_barrier()
            pltpu.sync_copy(
                acc.at[pl.ds(s * STRIPE, STRIPE)],
                out_hbm.at[c, sl, pl.ds(s * STRIPE, STRIPE)])

    return _prop


_prop1 = _make_prop(1)
_prop4 = _make_prop(4)


_TC_GRID = 16
NF = NP * 16 // 128
_FB = NF // _TC_GRID
_RB = STRIPE


def _frow_spec():
    return pl.BlockSpec((_FB, 128), lambda i: (i, 0))


def _fpair_spec():
    return pl.BlockSpec((NCORES, _FB, 128), lambda i: (0, i, 0))


def _full_spec(r, c):
    return pl.BlockSpec((r, c), lambda i: (0, 0))


_TCP = pltpu.CompilerParams(vmem_limit_bytes=100 * 2**20)


def _tc1a_body(deg_ref, dinv_ref):
    dinv_ref[...] = lax.rsqrt(deg_ref[0] + deg_ref[1] + 1.0)


def _tc1a(deg2f):
    return pl.pallas_call(
        _tc1a_body,
        grid=(1,),
        compiler_params=_TCP,
        in_specs=[pl.BlockSpec((NCORES, NF // 16, 128),
                               lambda i: (0, 0, 0))],
        out_specs=pl.BlockSpec((NF // 16, 128), lambda i: (0, 0)),
        out_shape=jax.ShapeDtypeStruct((NF // 16, 128), jnp.float32),
    )(deg2f)


def _tc1b_body(dinv_ref, xf_ref, g1_ref):
    g1_ref[...] = dinv_ref[...] * xf_ref[...]


def _tc1b(dinvf, xf):
    return pl.pallas_call(
        _tc1b_body,
        grid=(_TC_GRID,),
        compiler_params=_TCP,
        in_specs=[_frow_spec(), _frow_spec()],
        out_specs=_frow_spec(),
        out_shape=jax.ShapeDtypeStruct((NF, 128), jnp.float32),
    )(dinvf, xf)


def _tc2_body(acc1_ref, g1_ref, dinv_ref, w1_ref, b1_ref, *g2_refs):
    dinv = dinv_ref[...]
    p1 = dinv * (acc1_ref[0] + acc1_ref[1] + g1_ref[...])
    mid = jnp.maximum(
        jnp.dot(p1, w1_ref[...], preferred_element_type=jnp.float32,
                precision=lax.Precision.HIGHEST) + b1_ref[...], 0.0)
    for sl in range(4):
        g2_refs[sl][...] = dinv * mid[:, 128 * sl:128 * (sl + 1)]


def _tc2(acc1f, g1f, dinvf, w1perm, b1perm):
    return pl.pallas_call(
        _tc2_body,
        grid=(_TC_GRID,),
        compiler_params=_TCP,
        in_specs=[_fpair_spec(), _frow_spec(), _frow_spec(),
                  _full_spec(128, 512), _full_spec(1, 512)],
        out_specs=[_frow_spec()] * 4,
        out_shape=[jax.ShapeDtypeStruct((NF, 128), jnp.float32)] * 4,
    )(acc1f, g1f, dinvf, w1perm, b1perm)


def _tc3_body(acc2_ref, g2a_ref, g2b_ref, g2c_ref, g2d_ref, dinv_ref,
              w2_ref, b2_ref, w3_ref, g3_ref):
    dinv = dinv_ref[...]
    g2r = (g2a_ref, g2b_ref, g2c_ref, g2d_ref)
    p2 = jnp.concatenate(
        [dinv * (acc2_ref[0, sl] + acc2_ref[1, sl] + g2r[sl][...])
         for sl in range(4)], axis=1)
    mid = jnp.maximum(
        jnp.dot(p2, w2_ref[...], preferred_element_type=jnp.float32,
                precision=lax.Precision.HIGHEST) + b2_ref[...], 0.0)
    t3 = jnp.dot(mid, w3_ref[...], preferred_element_type=jnp.float32,
                 precision=lax.Precision.HIGHEST)
    g3_ref[...] = dinv * t3


def _tc3(acc2f, g2fs, dinvf, w2perm, b2perm, w3perm):
    return pl.pallas_call(
        _tc3_body,
        grid=(_TC_GRID,),
        compiler_params=_TCP,
        in_specs=[pl.BlockSpec((NCORES, 4, _FB, 128), lambda i: (0, 0, i, 0))]
                 + [_frow_spec()] * 4
                 + [_frow_spec(), _full_spec(512, 512), _full_spec(1, 512),
                    _full_spec(512, 128)],
        out_specs=_frow_spec(),
        out_shape=jax.ShapeDtypeStruct((NF, 128), jnp.float32),
    )(acc2f, *g2fs, dinvf, w2perm, b2perm, w3perm)


def _row_spec(cols):
    return pl.BlockSpec((_RB, cols), lambda i: (i, 0))


def _tc4_body(acc3_ref, g3_ref, dinv_ref, b3_ref, batch_ref, out_ref,
              sums_ref, cnts_ref):
    i = pl.program_id(0)

    @pl.when(i == 0)
    def _():
        sums_ref[...] = jnp.zeros((G, 16), jnp.float32)
        cnts_ref[...] = jnp.zeros((G, 16), jnp.float32)

    o = dinv_ref[...] * (acc3_ref[0] + acc3_ref[1] + g3_ref[...]) \
        + b3_ref[...]
    b = batch_ref[...][:, 0:1]
    onehot = (b == lax.broadcasted_iota(jnp.int32, (_RB, G), 1)
              ).astype(jnp.float32)
    sums_ref[...] += lax.dot_general(
        onehot, o, (((0,), (0,)), ((), ())),
        preferred_element_type=jnp.float32,
        precision=lax.Precision.HIGHEST)
    cnts_ref[...] += lax.dot_general(
        onehot, jnp.ones((_RB, 16), jnp.float32), (((0,), (0,)), ((), ())),
        preferred_element_type=jnp.float32,
        precision=lax.Precision.HIGHEST)

    @pl.when(i == _TC_GRID - 1)
    def _():
        out_ref[...] = sums_ref[...] / jnp.maximum(cnts_ref[...], 1.0)


def _tc4(acc3, g3, dinv16, b3p, batch2d):
    return pl.pallas_call(
        _tc4_body,
        grid=(_TC_GRID,),
        compiler_params=_TCP,
        in_specs=[pl.BlockSpec((NCORES, _RB, 16), lambda i: (0, i, 0)),
                  _row_spec(16), _row_spec(16), _full_spec(1, 16),
                  _row_spec(16)],
        out_specs=[_full_spec(G, 16)],
        out_shape=[jax.ShapeDtypeStruct((G, 16), jnp.float32)],
        scratch_shapes=[pltpu.VMEM((G, 16), jnp.float32),
                        pltpu.VMEM((G, 16), jnp.float32)],
    )(acc3, g3, dinv16, b3p, batch2d)


@jax.jit
def _run(x, edge_index, batch, W1, b1, W2, b2, W3, b3):
    src = edge_index[0].astype(jnp.int32)
    dst = edge_index[1].astype(jnp.int32)
    pad = N + (jnp.arange(EPAD - E, dtype=jnp.int32) % (NP - N))
    srcp = jnp.concatenate([src, pad]).reshape(NCORES, 16, NCHUNK, NJ, 128)
    dstp = jnp.concatenate([dst, pad]).reshape(NCORES, 16, NCHUNK, NJ, 128)

    xf = jnp.pad(x, ((0, NP - N), (0, 13))).reshape(NF, 128)
    eye8 = jnp.eye(8, dtype=jnp.float32)
    w1p = jnp.pad(W1, ((0, 13), (0, 0)))
    w1perm = jnp.einsum('kK,ism->kisKm', eye8,
                        w1p.reshape(16, 4, 16)).reshape(128, 512)
    w2perm = jnp.einsum('kK,amAM->akmAKM', eye8,
                        W2.reshape(4, 16, 4, 16)).reshape(512, 512)
    w3p = jnp.pad(W3, ((0, 0), (0, 11)))
    w3perm = jnp.einsum('kK,amM->akmKM', eye8,
                        w3p.reshape(4, 16, 16)).reshape(512, 128)
    b1perm = jnp.broadcast_to(b1.reshape(4, 1, 16), (4, 8, 16)).reshape(1, 512)
    b2perm = jnp.broadcast_to(b2.reshape(4, 1, 16), (4, 8, 16)).reshape(1, 512)
    b3p = jnp.pad(b3, (0, 11)).reshape(1, 16)
    batchp = jnp.concatenate(
        [batch.astype(jnp.int32),
         jnp.full((NP - N,), 99, jnp.int32)])
    batch2d = jnp.broadcast_to(batchp[:, None], (NP, 16))
    zrs = jnp.zeros((STRIPE, 16), jnp.float32)

    deg2 = _deg_kernel(dstp)
    dinv_lin = _tc1a(deg2.reshape(NCORES, NF // 16, 128))
    dinv16 = jnp.broadcast_to(dinv_lin.reshape(NP, 1), (NP, 16))
    dinvf = dinv16.reshape(NF, 128)
    g1f = _tc1b(dinvf, xf)
    acc1 = _prop1(g1f.reshape(NP, 16), srcp, dstp, zrs)
    g2fs = _tc2(acc1.reshape(NCORES, NF, 128), g1f, dinvf, w1perm, b1perm)
    acc2 = _prop4(*[g.reshape(NP, 16) for g in g2fs], srcp, dstp, zrs)
    g3f = _tc3(acc2.reshape(NCORES, 4, NF, 128), g2fs, dinvf,
               w2perm, b2perm, w3perm)
    acc3 = _prop1(g3f.reshape(NP, 16), srcp, dstp, zrs)
    (pooled,) = _tc4(acc3.reshape(NCORES, NP, 16), g3f.reshape(NP, 16),
                     dinv16, b3p, batch2d)
    return pooled[:, :5]


def kernel(x, edge_index, batch, W1, b1, W2, b2, W3, b3):
    return _run(x, edge_index, batch, W1, b1, W2, b2, W3, b3)


# speedup vs baseline: 37.4296x; 1.1582x over previous
import functools

import jax
import jax.numpy as jnp
from jax import lax
from jax.experimental import pallas as pl
from jax.experimental.pallas import tpu as pltpu
from jax.experimental.pallas import tpu_sc as plsc

N = 100000
E = 1600000
G = 32
NP = 100352
STRIPE = NP // 16
ZROWS = STRIPE // 8
NCORES = 2
NJ = 5
NCHUNK = 80
EPT = NCHUNK * NJ * 128
EPAD = NCORES * 16 * EPT


def _sc_mesh():
    return plsc.VectorSubcoreMesh(core_axis_name="c", subcore_axis_name="s")


@functools.partial(
    pl.kernel,
    out_type=jax.ShapeDtypeStruct((NCORES, NP), jnp.float32),
    mesh=_sc_mesh(),
    compiler_params=pltpu.CompilerParams(use_tc_tiling_on_sc=False),
    scratch_types=[
        pltpu.VMEM((2, NJ, 128), jnp.int32),
        pltpu.VMEM((128,), jnp.float32),
        pltpu.VMEM((ZROWS,), jnp.float32),
        pltpu.VMEM_SHARED((NP,), jnp.float32),
        pltpu.SemaphoreType.DMA,
        pltpu.SemaphoreType.DMA,
    ],
)
def _deg_kernel(dst_hbm, out_hbm, dstv, ones_v, zb, acc, sem0, sem1):
    c = lax.axis_index("c")
    s = lax.axis_index("s")
    sems = (sem0, sem1)

    for i in range(8):
        ones_v[pl.ds(16 * i, 16)] = jnp.ones((16,), jnp.float32)

    def zinit(i, carry):
        zb[pl.ds(16 * i, 16)] = jnp.zeros((16,), jnp.float32)
        return carry

    lax.fori_loop(0, ZROWS // 16, zinit, 0)
    for k in range(8):
        pltpu.sync_copy(zb, acc.at[pl.ds(s * STRIPE + k * ZROWS, ZROWS)])
    plsc.subcore_barrier()

    def idx(p, ci):
        pltpu.sync_copy(dst_hbm.at[c, s, ci], dstv.at[p])

    def fire_s(p):
        for j in range(NJ):
            pltpu.async_copy(ones_v, acc.at[dstv.at[p, j]], sems[p],
                             add=True)

    def drain_s(p):
        for j in range(NJ):
            pltpu.make_async_copy(ones_v, acc.at[dstv.at[p, j]],
                                  sems[p]).wait()

    idx(0, 0)

    def pair(i, carry):
        fire_s(0)
        idx(1, 2 * i + 1)
        fire_s(1)
        drain_s(0)
        idx(0, 2 * i + 2)
        drain_s(1)
        return carry

    lax.fori_loop(0, NCHUNK // 2 - 1, pair, 0)
    fire_s(0)
    idx(1, NCHUNK - 1)
    fire_s(1)
    drain_s(0)
    drain_s(1)
    plsc.subcore_barrier()
    pltpu.sync_copy(acc.at[pl.ds(s * STRIPE, STRIPE)],
                    out_hbm.at[c, pl.ds(s * STRIPE, STRIPE)])


def _make_prop(nsl):
    @functools.partial(
        pl.kernel,
        out_type=jax.ShapeDtypeStruct((NCORES, nsl, NP, 16), jnp.float32),
        mesh=_sc_mesh(),
        compiler_params=pltpu.CompilerParams(use_tc_tiling_on_sc=False),
        scratch_types=[
            pltpu.VMEM((2, NJ, 128), jnp.int32),
            pltpu.VMEM((2, NJ, 128), jnp.int32),
            pltpu.VMEM((2, NJ * 128, 16), jnp.float32),
            pltpu.VMEM_SHARED((NP, 16), jnp.float32),
            pltpu.SemaphoreType.DMA,
            pltpu.SemaphoreType.DMA,
        ],
    )
    def _prop(*refs):
        gs = refs[0:nsl]
        src_hbm, dst_hbm = refs[nsl], refs[nsl + 1]
        zeros_hbm, out_hbm = refs[nsl + 2], refs[nsl + 3]
        srcv, dstv, rows, acc = refs[nsl + 4:nsl + 8]
        semg = refs[nsl + 8:nsl + 10]
        c = lax.axis_index("c")
        s = lax.axis_index("s")

        for sl in range(nsl):
            g = gs[sl]

            def fire_g(p, ci):
                pltpu.sync_copy(src_hbm.at[c, s, ci], srcv.at[p])
                pltpu.sync_copy(dst_hbm.at[c, s, ci], dstv.at[p])
                for j in range(NJ):
                    pltpu.async_copy(
                        g.at[srcv.at[p, j]],
                        rows.at[p, pl.ds(128 * j, 128), :], semg[p])

            def drain_g(p):
                for j in range(NJ):
                    pltpu.make_async_copy(
                        g.at[srcv.at[p, j]],
                        rows.at[p, pl.ds(128 * j, 128), :], semg[p]).wait()

            def scat(p):
                for j in range(NJ):
                    pltpu.sync_copy(
                        rows.at[p, pl.ds(128 * j, 128), :],
                        acc.at[dstv.at[p, j]], add=True)

            pltpu.sync_copy(zeros_hbm, acc.at[pl.ds(s * STRIPE, STRIPE)])
            plsc.subcore_barrier()

            fire_g(0, 0)

            def pair(i, carry):
                fire_g(1, 2 * i + 1)
                drain_g(0)
                scat(0)
                fire_g(0, 2 * i + 2)
                drain_g(1)
                scat(1)
                return carry

            lax.fori_loop(0, NCHUNK // 2 - 1, pair, 0)
            fire_g(1, NCHUNK - 1)
            drain_g(0)
            scat(0)
            drain_g(1)
            scat(1)

            plsc.subcore_barrier()
            pltpu.sync_copy(
                acc.at[pl.ds(s * STRIPE, STRIPE)],
                out_hbm.at[c, sl, pl.ds(s * STRIPE, STRIPE)])

    return _prop


_prop1 = _make_prop(1)
_prop4 = _make_prop(4)


_TC_GRID = 16
NF = NP * 16 // 128
_FB = NF // _TC_GRID
_RB = STRIPE


def _frow_spec():
    return pl.BlockSpec((_FB, 128), lambda i: (i, 0))


def _fpair_spec():
    return pl.BlockSpec((NCORES, _FB, 128), lambda i: (0, i, 0))


def _full_spec(r, c):
    return pl.BlockSpec((r, c), lambda i: (0, 0))


_TCP = pltpu.CompilerParams(vmem_limit_bytes=100 * 2**20)


def _tc1a_body(deg_ref, dinv_ref):
    dinv_ref[...] = lax.rsqrt(deg_ref[0] + deg_ref[1] + 1.0)


def _tc1a(deg2f):
    return pl.pallas_call(
        _tc1a_body,
        grid=(1,),
        compiler_params=_TCP,
        in_specs=[pl.BlockSpec((NCORES, NF // 16, 128),
                               lambda i: (0, 0, 0))],
        out_specs=pl.BlockSpec((NF // 16, 128), lambda i: (0, 0)),
        out_shape=jax.ShapeDtypeStruct((NF // 16, 128), jnp.float32),
    )(deg2f)


def _tc1b_body(dinv_ref, xf_ref, g1_ref):
    g1_ref[...] = dinv_ref[...] * xf_ref[...]


def _tc1b(dinvf, xf):
    return pl.pallas_call(
        _tc1b_body,
        grid=(_TC_GRID,),
        compiler_params=_TCP,
        in_specs=[_frow_spec(), _frow_spec()],
        out_specs=_frow_spec(),
        out_shape=jax.ShapeDtypeStruct((NF, 128), jnp.float32),
    )(dinvf, xf)


def _tc2_body(acc1_ref, g1_ref, dinv_ref, w1_ref, b1_ref, *g2_refs):
    dinv = dinv_ref[...]
    p1 = dinv * (acc1_ref[0] + acc1_ref[1] + g1_ref[...])
    mid = jnp.maximum(
        jnp.dot(p1, w1_ref[...], preferred_element_type=jnp.float32,
                precision=lax.Precision.HIGHEST) + b1_ref[...], 0.0)
    for sl in range(4):
        g2_refs[sl][...] = dinv * mid[:, 128 * sl:128 * (sl + 1)]


def _tc2(acc1f, g1f, dinvf, w1perm, b1perm):
    return pl.pallas_call(
        _tc2_body,
        grid=(_TC_GRID,),
        compiler_params=_TCP,
        in_specs=[_fpair_spec(), _frow_spec(), _frow_spec(),
                  _full_spec(128, 512), _full_spec(1, 512)],
        out_specs=[_frow_spec()] * 4,
        out_shape=[jax.ShapeDtypeStruct((NF, 128), jnp.float32)] * 4,
    )(acc1f, g1f, dinvf, w1perm, b1perm)


def _tc3_body(acc2_ref, g2a_ref, g2b_ref, g2c_ref, g2d_ref, dinv_ref,
              w2_ref, b2_ref, w3_ref, g3_ref):
    dinv = dinv_ref[...]
    g2r = (g2a_ref, g2b_ref, g2c_ref, g2d_ref)
    p2 = jnp.concatenate(
        [dinv * (acc2_ref[0, sl] + acc2_ref[1, sl] + g2r[sl][...])
         for sl in range(4)], axis=1)
    mid = jnp.maximum(
        jnp.dot(p2, w2_ref[...], preferred_element_type=jnp.float32,
                precision=lax.Precision.HIGHEST) + b2_ref[...], 0.0)
    t3 = jnp.dot(mid, w3_ref[...], preferred_element_type=jnp.float32,
                 precision=lax.Precision.HIGHEST)
    g3_ref[...] = dinv * t3


def _tc3(acc2f, g2fs, dinvf, w2perm, b2perm, w3perm):
    return pl.pallas_call(
        _tc3_body,
        grid=(_TC_GRID,),
        compiler_params=_TCP,
        in_specs=[pl.BlockSpec((NCORES, 4, _FB, 128), lambda i: (0, 0, i, 0))]
                 + [_frow_spec()] * 4
                 + [_frow_spec(), _full_spec(512, 512), _full_spec(1, 512),
                    _full_spec(512, 128)],
        out_specs=_frow_spec(),
        out_shape=jax.ShapeDtypeStruct((NF, 128), jnp.float32),
    )(acc2f, *g2fs, dinvf, w2perm, b2perm, w3perm)


def _tc4_body(acc3_ref, g3_ref, dinv_ref, b3_ref, oh_ref, km_ref,
              f1_ref, f2_ref, out_ref, sums_ref, cnts_ref):
    i = pl.program_id(0)

    @pl.when(i == 0)
    def _():
        sums_ref[...] = jnp.zeros((G * 8, 128), jnp.float32)
        cnts_ref[...] = jnp.zeros((G * 8, 128), jnp.float32)

    of = dinv_ref[...] * (acc3_ref[0] + acc3_ref[1] + g3_ref[...]) \
        + b3_ref[...]
    oh = oh_ref[...]
    sums_ref[...] += lax.dot_general(
        oh, of, (((0,), (0,)), ((), ())),
        preferred_element_type=jnp.float32,
        precision=lax.Precision.HIGHEST)
    cnts_ref[...] += lax.dot_general(
        oh, jnp.ones((_FB, 128), jnp.float32), (((0,), (0,)), ((), ())),
        preferred_element_type=jnp.float32,
        precision=lax.Precision.HIGHEST)

    @pl.when(i == _TC_GRID - 1)
    def _():
        km = km_ref[...]
        f1 = f1_ref[...]
        f2 = f2_ref[...]
        sE = lax.dot_general(
            f2, jnp.dot(sums_ref[...] * km, f1,
                        preferred_element_type=jnp.float32,
                        precision=lax.Precision.HIGHEST),
            (((0,), (0,)), ((), ())),
            preferred_element_type=jnp.float32,
            precision=lax.Precision.HIGHEST)
        cE = lax.dot_general(
            f2, jnp.dot(cnts_ref[...] * km, f1,
                        preferred_element_type=jnp.float32,
                        precision=lax.Precision.HIGHEST),
            (((0,), (0,)), ((), ())),
            preferred_element_type=jnp.float32,
            precision=lax.Precision.HIGHEST)
        out_ref[...] = sE / jnp.maximum(cE, 1.0)


def _tc4(acc3f, g3f, dinvf, b3rep, onehotB, kmask, fold1, fold2):
    return pl.pallas_call(
        _tc4_body,
        grid=(_TC_GRID,),
        compiler_params=_TCP,
        in_specs=[_fpair_spec(), _frow_spec(), _frow_spec(),
                  _full_spec(1, 128),
                  pl.BlockSpec((_FB, G * 8), lambda i: (i, 0)),
                  _full_spec(G * 8, 128), _full_spec(128, 16),
                  _full_spec(G * 8, G)],
        out_specs=[_full_spec(G, 16)],
        out_shape=[jax.ShapeDtypeStruct((G, 16), jnp.float32)],
        scratch_shapes=[pltpu.VMEM((G * 8, 128), jnp.float32),
                        pltpu.VMEM((G * 8, 128), jnp.float32)],
    )(acc3f, g3f, dinvf, b3rep, onehotB, kmask, fold1, fold2)


@jax.jit
def _run(x, edge_index, batch, W1, b1, W2, b2, W3, b3):
    src = edge_index[0].astype(jnp.int32)
    dst = edge_index[1].astype(jnp.int32)
    pad = N + (jnp.arange(EPAD - E, dtype=jnp.int32) % (NP - N))
    srcp = jnp.concatenate([src, pad]).reshape(NCORES, 16, NCHUNK, NJ, 128)
    dstp = jnp.concatenate([dst, pad]).reshape(NCORES, 16, NCHUNK, NJ, 128)

    xf = jnp.pad(x, ((0, NP - N), (0, 13))).reshape(NF, 128)
    eye8 = jnp.eye(8, dtype=jnp.float32)
    w1p = jnp.pad(W1, ((0, 13), (0, 0)))
    w1perm = jnp.einsum('kK,ism->kisKm', eye8,
                        w1p.reshape(16, 4, 16)).reshape(128, 512)
    w2perm = jnp.einsum('kK,amAM->akmAKM', eye8,
                        W2.reshape(4, 16, 4, 16)).reshape(512, 512)
    w3p = jnp.pad(W3, ((0, 0), (0, 11)))
    w3perm = jnp.einsum('kK,amM->akmKM', eye8,
                        w3p.reshape(4, 16, 16)).reshape(512, 128)
    b1perm = jnp.broadcast_to(b1.reshape(4, 1, 16), (4, 8, 16)).reshape(1, 512)
    b2perm = jnp.broadcast_to(b2.reshape(4, 1, 16), (4, 8, 16)).reshape(1, 512)
    b3rep = jnp.broadcast_to(jnp.pad(b3, (0, 11)).reshape(1, 16),
                             (8, 16)).reshape(1, 128)
    batchp = jnp.concatenate(
        [batch.astype(jnp.int32),
         jnp.full((NP - N,), 99, jnp.int32)])
    onehotB = (batchp.reshape(NF, 1, 8) == jnp.arange(G, dtype=jnp.int32)
               .reshape(1, G, 1)).astype(jnp.float32).reshape(NF, G * 8)
    kmask = ((jnp.arange(G * 8, dtype=jnp.int32) % 8)[:, None] ==
             (jnp.arange(128, dtype=jnp.int32) // 16)[None, :]
             ).astype(jnp.float32)
    fold1 = ((jnp.arange(128, dtype=jnp.int32) % 16)[:, None] ==
             jnp.arange(16, dtype=jnp.int32)[None, :]).astype(jnp.float32)
    fold2 = ((jnp.arange(G * 8, dtype=jnp.int32) // 8)[:, None] ==
             jnp.arange(G, dtype=jnp.int32)[None, :]).astype(jnp.float32)
    zrs = jnp.zeros((STRIPE, 16), jnp.float32)

    deg2 = _deg_kernel(dstp)
    dinv_lin = _tc1a(deg2.reshape(NCORES, NF // 16, 128))
    dinvf = jnp.broadcast_to(
        dinv_lin.reshape(NP, 1), (NP, 16)).reshape(NF, 128)
    g1f = _tc1b(dinvf, xf)
    acc1 = _prop1(g1f.reshape(NP, 16), srcp, dstp, zrs)
    g2fs = _tc2(acc1.reshape(NCORES, NF, 128), g1f, dinvf, w1perm, b1perm)
    acc2 = _prop4(*[g.reshape(NP, 16) for g in g2fs], srcp, dstp, zrs)
    g3f = _tc3(acc2.reshape(NCORES, 4, NF, 128), g2fs, dinvf,
               w2perm, b2perm, w3perm)
    acc3 = _prop1(g3f.reshape(NP, 16), srcp, dstp, zrs)
    (pooled,) = _tc4(acc3.reshape(NCORES, NF, 128), g3f, dinvf, b3rep,
                     onehotB, kmask, fold1, fold2)
    return pooled[:, :5]


def kernel(x, edge_index, batch, W1, b1, W2, b2, W3, b3):
    return _run(x, edge_index, batch, W1, b1, W2, b2, W3, b3)


# speedup vs baseline: 38.0782x; 1.0173x over previous
import functools

import jax
import jax.numpy as jnp
from jax import lax
from jax.experimental import pallas as pl
from jax.experimental.pallas import tpu as pltpu
from jax.experimental.pallas import tpu_sc as plsc

N = 100000
E = 1600000
G = 32
NP = 100352
STRIPE = NP // 16
ZROWS = STRIPE // 8
NCORES = 2
NJ = 5
NCHUNK = 80
EPT = NCHUNK * NJ * 128
EPAD = NCORES * 16 * EPT


def _sc_mesh():
    return plsc.VectorSubcoreMesh(core_axis_name="c", subcore_axis_name="s")


@functools.partial(
    pl.kernel,
    out_type=jax.ShapeDtypeStruct((NCORES, NP), jnp.float32),
    mesh=_sc_mesh(),
    compiler_params=pltpu.CompilerParams(use_tc_tiling_on_sc=False),
    scratch_types=[
        pltpu.VMEM((2, NJ, 128), jnp.int32),
        pltpu.VMEM((128,), jnp.float32),
        pltpu.VMEM((ZROWS,), jnp.float32),
        pltpu.VMEM_SHARED((NP,), jnp.float32),
        pltpu.SemaphoreType.DMA,
        pltpu.SemaphoreType.DMA,
    ],
)
def _deg_kernel(ei_hbm, out_hbm, dstv, ones_v, zb, acc, sem0, sem1):
    c = lax.axis_index("c")
    s = lax.axis_index("s")
    sems = (sem0, sem1)

    for i in range(8):
        ones_v[pl.ds(16 * i, 16)] = jnp.ones((16,), jnp.float32)

    def zinit(i, carry):
        zb[pl.ds(16 * i, 16)] = jnp.zeros((16,), jnp.float32)
        return carry

    lax.fori_loop(0, ZROWS // 16, zinit, 0)
    for k in range(8):
        pltpu.sync_copy(zb, acc.at[pl.ds(s * STRIPE + k * ZROWS, ZROWS)])
    plsc.subcore_barrier()

    def idx(p, ci):
        pltpu.sync_copy(ei_hbm.at[1, c, s, ci], dstv.at[p])

    def fire_s(p):
        for j in range(NJ):
            pltpu.async_copy(ones_v, acc.at[dstv.at[p, j]], sems[p],
                             add=True)

    def drain_s(p):
        for j in range(NJ):
            pltpu.make_async_copy(ones_v, acc.at[dstv.at[p, j]],
                                  sems[p]).wait()

    idx(0, 0)

    def pair(i, carry):
        fire_s(0)
        idx(1, 2 * i + 1)
        fire_s(1)
        drain_s(0)
        idx(0, 2 * i + 2)
        drain_s(1)
        return carry

    lax.fori_loop(0, NCHUNK // 2 - 1, pair, 0)
    fire_s(0)
    idx(1, NCHUNK - 1)
    fire_s(1)
    drain_s(0)
    drain_s(1)
    plsc.subcore_barrier()
    pltpu.sync_copy(acc.at[pl.ds(s * STRIPE, STRIPE)],
                    out_hbm.at[c, pl.ds(s * STRIPE, STRIPE)])


def _make_prop(nsl):
    @functools.partial(
        pl.kernel,
        out_type=jax.ShapeDtypeStruct((NCORES, nsl, NP, 16), jnp.float32),
        mesh=_sc_mesh(),
        compiler_params=pltpu.CompilerParams(use_tc_tiling_on_sc=False),
        scratch_types=[
            pltpu.VMEM((2, NJ, 128), jnp.int32),
            pltpu.VMEM((2, NJ, 128), jnp.int32),
            pltpu.VMEM((2, NJ * 128, 16), jnp.float32),
            pltpu.VMEM_SHARED((NP, 16), jnp.float32),
            pltpu.SemaphoreType.DMA,
            pltpu.SemaphoreType.DMA,
        ],
    )
    def _prop(*refs):
        gs = refs[0:nsl]
        ei_hbm = refs[nsl]
        zeros_hbm, out_hbm = refs[nsl + 1], refs[nsl + 2]
        srcv, dstv, rows, acc = refs[nsl + 3:nsl + 7]
        semg = refs[nsl + 7:nsl + 9]
        c = lax.axis_index("c")
        s = lax.axis_index("s")

        for sl in range(nsl):
            g = gs[sl]

            def fire_g(p, ci):
                pltpu.sync_copy(ei_hbm.at[0, c, s, ci], srcv.at[p])
                pltpu.sync_copy(ei_hbm.at[1, c, s, ci], dstv.at[p])
                for j in range(NJ):
                    pltpu.async_copy(
                        g.at[srcv.at[p, j]],
                        rows.at[p, pl.ds(128 * j, 128), :], semg[p])

            def drain_g(p):
                for j in range(NJ):
                    pltpu.make_async_copy(
                        g.at[srcv.at[p, j]],
                        rows.at[p, pl.ds(128 * j, 128), :], semg[p]).wait()

            def scat(p):
                for j in range(NJ):
                    pltpu.sync_copy(
                        rows.at[p, pl.ds(128 * j, 128), :],
                        acc.at[dstv.at[p, j]], add=True)

            pltpu.sync_copy(zeros_hbm, acc.at[pl.ds(s * STRIPE, STRIPE)])
            plsc.subcore_barrier()

            fire_g(0, 0)

            def pair(i, carry):
                fire_g(1, 2 * i + 1)
                drain_g(0)
                scat(0)
                fire_g(0, 2 * i + 2)
                drain_g(1)
                scat(1)
                return carry

            lax.fori_loop(0, NCHUNK // 2 - 1, pair, 0)
            fire_g(1, NCHUNK - 1)
            drain_g(0)
            scat(0)
            drain_g(1)
            scat(1)

            plsc.subcore_barrier()
            pltpu.sync_copy(
                acc.at[pl.ds(s * STRIPE, STRIPE)],
                out_hbm.at[c, sl, pl.ds(s * STRIPE, STRIPE)])

    return _prop


_prop1 = _make_prop(1)
_prop4 = _make_prop(4)


_TC_GRID = 16
NF = NP * 16 // 128
_FB = NF // _TC_GRID
_RB = STRIPE


def _frow_spec():
    return pl.BlockSpec((_FB, 128), lambda i: (i, 0))


def _fpair_spec():
    return pl.BlockSpec((NCORES, _FB, 128), lambda i: (0, i, 0))


def _full_spec(r, c):
    return pl.BlockSpec((r, c), lambda i: (0, 0))


_TCP = pltpu.CompilerParams(vmem_limit_bytes=100 * 2**20)


def _tc1a_body(deg_ref, dinv_ref):
    dinv_ref[...] = lax.rsqrt(deg_ref[0] + deg_ref[1] + 1.0)


def _tc1a(deg2f):
    return pl.pallas_call(
        _tc1a_body,
        grid=(1,),
        compiler_params=_TCP,
        in_specs=[pl.BlockSpec((NCORES, NF // 16, 128),
                               lambda i: (0, 0, 0))],
        out_specs=pl.BlockSpec((NF // 16, 128), lambda i: (0, 0)),
        out_shape=jax.ShapeDtypeStruct((NF // 16, 128), jnp.float32),
    )(deg2f)


def _tc1b_body(dinv_ref, xf_ref, g1_ref):
    g1_ref[...] = dinv_ref[...] * xf_ref[...]


def _tc1b(dinvf, xf):
    return pl.pallas_call(
        _tc1b_body,
        grid=(_TC_GRID,),
        compiler_params=_TCP,
        in_specs=[_frow_spec(), _frow_spec()],
        out_specs=_frow_spec(),
        out_shape=jax.ShapeDtypeStruct((NF, 128), jnp.float32),
    )(dinvf, xf)


def _tc2_body(acc1_ref, g1_ref, dinv_ref, w1_ref, b1_ref, *g2_refs):
    dinv = dinv_ref[...]
    p1 = dinv * (acc1_ref[0] + acc1_ref[1] + g1_ref[...])
    mid = jnp.maximum(
        jnp.dot(p1, w1_ref[...], preferred_element_type=jnp.float32,
                precision=lax.Precision.HIGHEST) + b1_ref[...], 0.0)
    for sl in range(4):
        g2_refs[sl][...] = dinv * mid[:, 128 * sl:128 * (sl + 1)]


def _tc2(acc1f, g1f, dinvf, w1perm, b1perm):
    return pl.pallas_call(
        _tc2_body,
        grid=(_TC_GRID,),
        compiler_params=_TCP,
        in_specs=[_fpair_spec(), _frow_spec(), _frow_spec(),
                  _full_spec(128, 512), _full_spec(1, 512)],
        out_specs=[_frow_spec()] * 4,
        out_shape=[jax.ShapeDtypeStruct((NF, 128), jnp.float32)] * 4,
    )(acc1f, g1f, dinvf, w1perm, b1perm)


def _tc3_body(acc2_ref, g2a_ref, g2b_ref, g2c_ref, g2d_ref, dinv_ref,
              w2_ref, b2_ref, w3_ref, g3_ref):
    dinv = dinv_ref[...]
    g2r = (g2a_ref, g2b_ref, g2c_ref, g2d_ref)
    p2 = jnp.concatenate(
        [dinv * (acc2_ref[0, sl] + acc2_ref[1, sl] + g2r[sl][...])
         for sl in range(4)], axis=1)
    mid = jnp.maximum(
        jnp.dot(p2, w2_ref[...], preferred_element_type=jnp.float32,
                precision=lax.Precision.HIGHEST) + b2_ref[...], 0.0)
    t3 = jnp.dot(mid, w3_ref[...], preferred_element_type=jnp.float32,
                 precision=lax.Precision.HIGHEST)
    g3_ref[...] = dinv * t3


def _tc3(acc2f, g2fs, dinvf, w2perm, b2perm, w3perm):
    return pl.pallas_call(
        _tc3_body,
        grid=(_TC_GRID,),
        compiler_params=_TCP,
        in_specs=[pl.BlockSpec((NCORES, 4, _FB, 128), lambda i: (0, 0, i, 0))]
                 + [_frow_spec()] * 4
                 + [_frow_spec(), _full_spec(512, 512), _full_spec(1, 512),
                    _full_spec(512, 128)],
        out_specs=_frow_spec(),
        out_shape=jax.ShapeDtypeStruct((NF, 128), jnp.float32),
    )(acc2f, *g2fs, dinvf, w2perm, b2perm, w3perm)


def _tc4_body(acc3_ref, g3_ref, dinv_ref, b3_ref, oh_ref, km_ref,
              f1_ref, f2_ref, out_ref, sums_ref, cnts_ref):
    i = pl.program_id(0)

    @pl.when(i == 0)
    def _():
        sums_ref[...] = jnp.zeros((G * 8, 128), jnp.float32)
        cnts_ref[...] = jnp.zeros((G * 8, 128), jnp.float32)

    of = dinv_ref[...] * (acc3_ref[0] + acc3_ref[1] + g3_ref[...]) \
        + b3_ref[...]
    oh = oh_ref[...]
    sums_ref[...] += lax.dot_general(
        oh, of, (((0,), (0,)), ((), ())),
        preferred_element_type=jnp.float32,
        precision=lax.Precision.HIGHEST)
    cnts_ref[...] += lax.dot_general(
        oh, jnp.ones((_FB, 128), jnp.float32), (((0,), (0,)), ((), ())),
        preferred_element_type=jnp.float32,
        precision=lax.Precision.HIGHEST)

    @pl.when(i == _TC_GRID - 1)
    def _():
        km = km_ref[...]
        f1 = f1_ref[...]
        f2 = f2_ref[...]
        sE = lax.dot_general(
            f2, jnp.dot(sums_ref[...] * km, f1,
                        preferred_element_type=jnp.float32,
                        precision=lax.Precision.HIGHEST),
            (((0,), (0,)), ((), ())),
            preferred_element_type=jnp.float32,
            precision=lax.Precision.HIGHEST)
        cE = lax.dot_general(
            f2, jnp.dot(cnts_ref[...] * km, f1,
                        preferred_element_type=jnp.float32,
                        precision=lax.Precision.HIGHEST),
            (((0,), (0,)), ((), ())),
            preferred_element_type=jnp.float32,
            precision=lax.Precision.HIGHEST)
        out_ref[...] = sE / jnp.maximum(cE, 1.0)


def _tc4(acc3f, g3f, dinvf, b3rep, onehotB, kmask, fold1, fold2):
    return pl.pallas_call(
        _tc4_body,
        grid=(_TC_GRID,),
        compiler_params=_TCP,
        in_specs=[_fpair_spec(), _frow_spec(), _frow_spec(),
                  _full_spec(1, 128),
                  pl.BlockSpec((_FB, G * 8), lambda i: (i, 0)),
                  _full_spec(G * 8, 128), _full_spec(128, 16),
                  _full_spec(G * 8, G)],
        out_specs=[_full_spec(G, 16)],
        out_shape=[jax.ShapeDtypeStruct((G, 16), jnp.float32)],
        scratch_shapes=[pltpu.VMEM((G * 8, 128), jnp.float32),
                        pltpu.VMEM((G * 8, 128), jnp.float32)],
    )(acc3f, g3f, dinvf, b3rep, onehotB, kmask, fold1, fold2)


@jax.jit
def _run(x, edge_index, batch, W1, b1, W2, b2, W3, b3):
    pad = N + (jnp.arange(EPAD - E, dtype=jnp.int32) % (NP - N))
    eip = jnp.concatenate(
        [edge_index.astype(jnp.int32),
         jnp.broadcast_to(pad[None, :], (2, EPAD - E))],
        axis=1).reshape(2, NCORES, 16, NCHUNK, NJ, 128)

    xf = jnp.pad(x, ((0, NP - N), (0, 13))).reshape(NF, 128)
    eye8 = jnp.eye(8, dtype=jnp.float32)
    w1p = jnp.pad(W1, ((0, 13), (0, 0)))
    w1perm = jnp.einsum('kK,ism->kisKm', eye8,
                        w1p.reshape(16, 4, 16)).reshape(128, 512)
    w2perm = jnp.einsum('kK,amAM->akmAKM', eye8,
                        W2.reshape(4, 16, 4, 16)).reshape(512, 512)
    w3p = jnp.pad(W3, ((0, 0), (0, 11)))
    w3perm = jnp.einsum('kK,amM->akmKM', eye8,
                        w3p.reshape(4, 16, 16)).reshape(512, 128)
    b1perm = jnp.broadcast_to(b1.reshape(4, 1, 16), (4, 8, 16)).reshape(1, 512)
    b2perm = jnp.broadcast_to(b2.reshape(4, 1, 16), (4, 8, 16)).reshape(1, 512)
    b3rep = jnp.broadcast_to(jnp.pad(b3, (0, 11)).reshape(1, 16),
                             (8, 16)).reshape(1, 128)
    batchp = jnp.concatenate(
        [batch.astype(jnp.int32),
         jnp.full((NP - N,), 99, jnp.int32)])
    onehotB = (batchp.reshape(NF, 1, 8) == jnp.arange(G, dtype=jnp.int32)
               .reshape(1, G, 1)).astype(jnp.float32).reshape(NF, G * 8)
    kmask = ((jnp.arange(G * 8, dtype=jnp.int32) % 8)[:, None] ==
             (jnp.arange(128, dtype=jnp.int32) // 16)[None, :]
             ).astype(jnp.float32)
    fold1 = ((jnp.arange(128, dtype=jnp.int32) % 16)[:, None] ==
             jnp.arange(16, dtype=jnp.int32)[None, :]).astype(jnp.float32)
    fold2 = ((jnp.arange(G * 8, dtype=jnp.int32) // 8)[:, None] ==
             jnp.arange(G, dtype=jnp.int32)[None, :]).astype(jnp.float32)
    zrs = jnp.zeros((STRIPE, 16), jnp.float32)

    deg2 = _deg_kernel(eip)
    dinv_lin = _tc1a(deg2.reshape(NCORES, NF // 16, 128))
    dinvf = jnp.broadcast_to(
        dinv_lin.reshape(NP, 1), (NP, 16)).reshape(NF, 128)
    g1f = _tc1b(dinvf, xf)
    acc1 = _prop1(g1f.reshape(NP, 16), eip, zrs)
    g2fs = _tc2(acc1.reshape(NCORES, NF, 128), g1f, dinvf, w1perm, b1perm)
    acc2 = _prop4(*[g.reshape(NP, 16) for g in g2fs], eip, zrs)
    g3f = _tc3(acc2.reshape(NCORES, 4, NF, 128), g2fs, dinvf,
               w2perm, b2perm, w3perm)
    acc3 = _prop1(g3f.reshape(NP, 16), eip, zrs)
    (pooled,) = _tc4(acc3.reshape(NCORES, NF, 128), g3f, dinvf, b3rep,
                     onehotB, kmask, fold1, fold2)
    return pooled[:, :5]


def kernel(x, edge_index, batch, W1, b1, W2, b2, W3, b3):
    return _run(x, edge_index, batch, W1, b1, W2, b2, W3, b3)


# speedup vs baseline: 39.8533x; 1.0466x over previous
import functools

import jax
import jax.numpy as jnp
from jax import lax
from jax.experimental import pallas as pl
from jax.experimental.pallas import tpu as pltpu
from jax.experimental.pallas import tpu_sc as plsc

N = 100000
E = 1600000
G = 32
NP = 100352
STRIPE = NP // 16
ZROWS = STRIPE // 8
NCORES = 2
NJ = 5
NCHUNK = 80
EPT = NCHUNK * NJ * 128
EPAD = NCORES * 16 * EPT


def _sc_mesh():
    return plsc.VectorSubcoreMesh(core_axis_name="c", subcore_axis_name="s")


@functools.partial(
    pl.kernel,
    out_type=jax.ShapeDtypeStruct((NCORES, NP), jnp.float32),
    mesh=_sc_mesh(),
    compiler_params=pltpu.CompilerParams(use_tc_tiling_on_sc=False),
    scratch_types=[
        pltpu.VMEM((2, NJ, 128), jnp.int32),
        pltpu.VMEM((128,), jnp.float32),
        pltpu.VMEM((ZROWS,), jnp.float32),
        pltpu.VMEM_SHARED((NP,), jnp.float32),
        pltpu.SemaphoreType.DMA,
        pltpu.SemaphoreType.DMA,
    ],
)
def _deg_kernel(ei_hbm, out_hbm, dstv, ones_v, zb, acc, sem0, sem1):
    c = lax.axis_index("c")
    s = lax.axis_index("s")
    sems = (sem0, sem1)

    for i in range(8):
        ones_v[pl.ds(16 * i, 16)] = jnp.ones((16,), jnp.float32)

    def zinit(i, carry):
        zb[pl.ds(16 * i, 16)] = jnp.zeros((16,), jnp.float32)
        return carry

    lax.fori_loop(0, ZROWS // 16, zinit, 0)
    for k in range(8):
        pltpu.sync_copy(zb, acc.at[pl.ds(s * STRIPE + k * ZROWS, ZROWS)])
    plsc.subcore_barrier()

    def idx(p, ci):
        pltpu.sync_copy(ei_hbm.at[1, c, s, ci], dstv.at[p])

    def fire_s(p):
        for j in range(NJ):
            pltpu.async_copy(ones_v, acc.at[dstv.at[p, j]], sems[p],
                             add=True)

    def drain_s(p):
        for j in range(NJ):
            pltpu.make_async_copy(ones_v, acc.at[dstv.at[p, j]],
                                  sems[p]).wait()

    idx(0, 0)

    def pair(i, carry):
        fire_s(0)
        idx(1, 2 * i + 1)
        fire_s(1)
        drain_s(0)
        idx(0, 2 * i + 2)
        drain_s(1)
        return carry

    lax.fori_loop(0, NCHUNK // 2 - 1, pair, 0)
    fire_s(0)
    idx(1, NCHUNK - 1)
    fire_s(1)
    drain_s(0)
    drain_s(1)
    plsc.subcore_barrier()
    pltpu.sync_copy(acc.at[pl.ds(s * STRIPE, STRIPE)],
                    out_hbm.at[c, pl.ds(s * STRIPE, STRIPE)])


def _make_prop(nsl):
    @functools.partial(
        pl.kernel,
        out_type=jax.ShapeDtypeStruct((NCORES, nsl, NP, 16), jnp.float32),
        mesh=_sc_mesh(),
        compiler_params=pltpu.CompilerParams(use_tc_tiling_on_sc=False),
        scratch_types=[
            pltpu.VMEM((2, NJ, 128), jnp.int32),
            pltpu.VMEM((2, NJ, 128), jnp.int32),
            pltpu.VMEM((2, NJ * 128, 16), jnp.float32),
            pltpu.VMEM_SHARED((NP, 16), jnp.float32),
            pltpu.SemaphoreType.DMA,
            pltpu.SemaphoreType.DMA,
            pltpu.SemaphoreType.DMA,
            pltpu.SemaphoreType.DMA,
        ],
    )
    def _prop(*refs):
        gs = refs[0:nsl]
        ei_hbm = refs[nsl]
        zeros_hbm, out_hbm = refs[nsl + 1], refs[nsl + 2]
        srcv, dstv, rows, acc = refs[nsl + 3:nsl + 7]
        semg = refs[nsl + 7:nsl + 9]
        sems = refs[nsl + 9:nsl + 11]
        c = lax.axis_index("c")
        s = lax.axis_index("s")

        for sl in range(nsl):
            g = gs[sl]

            def fire_g(p, ci):
                pltpu.sync_copy(ei_hbm.at[0, c, s, ci], srcv.at[p])
                pltpu.sync_copy(ei_hbm.at[1, c, s, ci], dstv.at[p])
                for j in range(NJ):
                    pltpu.async_copy(
                        g.at[srcv.at[p, j]],
                        rows.at[p, pl.ds(128 * j, 128), :], semg[p])

            def drain_g(p):
                for j in range(NJ):
                    pltpu.make_async_copy(
                        g.at[srcv.at[p, j]],
                        rows.at[p, pl.ds(128 * j, 128), :], semg[p]).wait()

            def fire_s(p):
                for j in range(NJ):
                    pltpu.async_copy(
                        rows.at[p, pl.ds(128 * j, 128), :],
                        acc.at[dstv.at[p, j]], sems[p], add=True)

            def drain_s(p):
                for j in range(NJ):
                    pltpu.make_async_copy(
                        rows.at[p, pl.ds(128 * j, 128), :],
                        acc.at[dstv.at[p, j]], sems[p]).wait()

            pltpu.sync_copy(zeros_hbm, acc.at[pl.ds(s * STRIPE, STRIPE)])
            plsc.subcore_barrier()

            fire_g(0, 0)

            def pair(i, carry):
                fire_g(1, 2 * i + 1)
                drain_g(0)
                fire_s(0)
                drain_g(1)
                fire_s(1)
                drain_s(0)
                fire_g(0, 2 * i + 2)
                drain_s(1)
                return carry

            lax.fori_loop(0, NCHUNK // 2 - 1, pair, 0)
            fire_g(1, NCHUNK - 1)
            drain_g(0)
            fire_s(0)
            drain_g(1)
            fire_s(1)
            drain_s(0)
            drain_s(1)

            plsc.subcore_barrier()
            pltpu.sync_copy(
                acc.at[pl.ds(s * STRIPE, STRIPE)],
                out_hbm.at[c, sl, pl.ds(s * STRIPE, STRIPE)])

    return _prop


_prop1 = _make_prop(1)
_prop4 = _make_prop(4)


_TC_GRID = 16
NF = NP * 16 // 128
_FB = NF // _TC_GRID
_RB = STRIPE


def _frow_spec():
    return pl.BlockSpec((_FB, 128), lambda i: (i, 0))


def _fpair_spec():
    return pl.BlockSpec((NCORES, _FB, 128), lambda i: (0, i, 0))


def _full_spec(r, c):
    return pl.BlockSpec((r, c), lambda i: (0, 0))


_TCP = pltpu.CompilerParams(vmem_limit_bytes=100 * 2**20)


def _tc1a_body(deg_ref, dinv_ref):
    dinv_ref[...] = lax.rsqrt(deg_ref[0] + deg_ref[1] + 1.0)


def _tc1a(deg2f):
    return pl.pallas_call(
        _tc1a_body,
        grid=(1,),
        compiler_params=_TCP,
        in_specs=[pl.BlockSpec((NCORES, NF // 16, 128),
                               lambda i: (0, 0, 0))],
        out_specs=pl.BlockSpec((NF // 16, 128), lambda i: (0, 0)),
        out_shape=jax.ShapeDtypeStruct((NF // 16, 128), jnp.float32),
    )(deg2f)


def _tc1b_body(dinv_ref, xf_ref, g1_ref):
    g1_ref[...] = dinv_ref[...] * xf_ref[...]


def _tc1b(dinvf, xf):
    return pl.pallas_call(
        _tc1b_body,
        grid=(_TC_GRID,),
        compiler_params=_TCP,
        in_specs=[_frow_spec(), _frow_spec()],
        out_specs=_frow_spec(),
        out_shape=jax.ShapeDtypeStruct((NF, 128), jnp.float32),
    )(dinvf, xf)


def _tc2_body(acc1_ref, g1_ref, dinv_ref, w1_ref, b1_ref, *g2_refs):
    dinv = dinv_ref[...]
    p1 = dinv * (acc1_ref[0] + acc1_ref[1] + g1_ref[...])
    mid = jnp.maximum(
        jnp.dot(p1, w1_ref[...], preferred_element_type=jnp.float32,
                precision=lax.Precision.DEFAULT) + b1_ref[...], 0.0)
    for sl in range(4):
        g2_refs[sl][...] = dinv * mid[:, 128 * sl:128 * (sl + 1)]


def _tc2(acc1f, g1f, dinvf, w1perm, b1perm):
    return pl.pallas_call(
        _tc2_body,
        grid=(_TC_GRID,),
        compiler_params=_TCP,
        in_specs=[_fpair_spec(), _frow_spec(), _frow_spec(),
                  _full_spec(128, 512), _full_spec(1, 512)],
        out_specs=[_frow_spec()] * 4,
        out_shape=[jax.ShapeDtypeStruct((NF, 128), jnp.float32)] * 4,
    )(acc1f, g1f, dinvf, w1perm, b1perm)


def _tc3_body(acc2_ref, g2a_ref, g2b_ref, g2c_ref, g2d_ref, dinv_ref,
              w2_ref, b2_ref, w3_ref, g3_ref):
    dinv = dinv_ref[...]
    g2r = (g2a_ref, g2b_ref, g2c_ref, g2d_ref)
    p2 = jnp.concatenate(
        [dinv * (acc2_ref[0, sl] + acc2_ref[1, sl] + g2r[sl][...])
         for sl in range(4)], axis=1)
    mid = jnp.maximum(
        jnp.dot(p2, w2_ref[...], preferred_element_type=jnp.float32,
                precision=lax.Precision.DEFAULT) + b2_ref[...], 0.0)
    t3 = jnp.dot(mid, w3_ref[...], preferred_element_type=jnp.float32,
                 precision=lax.Precision.DEFAULT)
    g3_ref[...] = dinv * t3


def _tc3(acc2f, g2fs, dinvf, w2perm, b2perm, w3perm):
    return pl.pallas_call(
        _tc3_body,
        grid=(_TC_GRID,),
        compiler_params=_TCP,
        in_specs=[pl.BlockSpec((NCORES, 4, _FB, 128), lambda i: (0, 0, i, 0))]
                 + [_frow_spec()] * 4
                 + [_frow_spec(), _full_spec(512, 512), _full_spec(1, 512),
                    _full_spec(512, 128)],
        out_specs=_frow_spec(),
        out_shape=jax.ShapeDtypeStruct((NF, 128), jnp.float32),
    )(acc2f, *g2fs, dinvf, w2perm, b2perm, w3perm)


def _tc4_body(acc3_ref, g3_ref, dinv_ref, b3_ref, oh_ref, km_ref,
              f1_ref, f2_ref, out_ref, sums_ref, cnts_ref):
    i = pl.program_id(0)

    @pl.when(i == 0)
    def _():
        sums_ref[...] = jnp.zeros((G * 8, 128), jnp.float32)
        cnts_ref[...] = jnp.zeros((G * 8, 128), jnp.float32)

    of = dinv_ref[...] * (acc3_ref[0] + acc3_ref[1] + g3_ref[...]) \
        + b3_ref[...]
    oh = oh_ref[...]
    sums_ref[...] += lax.dot_general(
        oh, of, (((0,), (0,)), ((), ())),
        preferred_element_type=jnp.float32,
        precision=lax.Precision.HIGHEST)
    cnts_ref[...] += lax.dot_general(
        oh, jnp.ones((_FB, 128), jnp.float32), (((0,), (0,)), ((), ())),
        preferred_element_type=jnp.float32,
        precision=lax.Precision.HIGHEST)

    @pl.when(i == _TC_GRID - 1)
    def _():
        km = km_ref[...]
        f1 = f1_ref[...]
        f2 = f2_ref[...]
        sE = lax.dot_general(
            f2, jnp.dot(sums_ref[...] * km, f1,
                        preferred_element_type=jnp.float32,
                        precision=lax.Precision.HIGHEST),
            (((0,), (0,)), ((), ())),
            preferred_element_type=jnp.float32,
            precision=lax.Precision.HIGHEST)
        cE = lax.dot_general(
            f2, jnp.dot(cnts_ref[...] * km, f1,
                        preferred_element_type=jnp.float32,
                        precision=lax.Precision.HIGHEST),
            (((0,), (0,)), ((), ())),
            preferred_element_type=jnp.float32,
            precision=lax.Precision.HIGHEST)
        out_ref[...] = sE / jnp.maximum(cE, 1.0)


def _tc4(acc3f, g3f, dinvf, b3rep, onehotB, kmask, fold1, fold2):
    return pl.pallas_call(
        _tc4_body,
        grid=(_TC_GRID,),
        compiler_params=_TCP,
        in_specs=[_fpair_spec(), _frow_spec(), _frow_spec(),
                  _full_spec(1, 128),
                  pl.BlockSpec((_FB, G * 8), lambda i: (i, 0)),
                  _full_spec(G * 8, 128), _full_spec(128, 16),
                  _full_spec(G * 8, G)],
        out_specs=[_full_spec(G, 16)],
        out_shape=[jax.ShapeDtypeStruct((G, 16), jnp.float32)],
        scratch_shapes=[pltpu.VMEM((G * 8, 128), jnp.float32),
                        pltpu.VMEM((G * 8, 128), jnp.float32)],
    )(acc3f, g3f, dinvf, b3rep, onehotB, kmask, fold1, fold2)


@jax.jit
def _run(x, edge_index, batch, W1, b1, W2, b2, W3, b3):
    pad = N + (jnp.arange(EPAD - E, dtype=jnp.int32) % (NP - N))
    eip = jnp.concatenate(
        [edge_index.astype(jnp.int32),
         jnp.broadcast_to(pad[None, :], (2, EPAD - E))],
        axis=1).reshape(2, NCORES, 16, NCHUNK, NJ, 128)

    xf = jnp.pad(x, ((0, NP - N), (0, 13))).reshape(NF, 128)
    eye8 = jnp.eye(8, dtype=jnp.float32)
    w1p = jnp.pad(W1, ((0, 13), (0, 0)))
    w1perm = jnp.einsum('kK,ism->kisKm', eye8,
                        w1p.reshape(16, 4, 16)).reshape(128, 512)
    w2perm = jnp.einsum('kK,amAM->akmAKM', eye8,
                        W2.reshape(4, 16, 4, 16)).reshape(512, 512)
    w3p = jnp.pad(W3, ((0, 0), (0, 11)))
    w3perm = jnp.einsum('kK,amM->akmKM', eye8,
                        w3p.reshape(4, 16, 16)).reshape(512, 128)
    b1perm = jnp.broadcast_to(b1.reshape(4, 1, 16), (4, 8, 16)).reshape(1, 512)
    b2perm = jnp.broadcast_to(b2.reshape(4, 1, 16), (4, 8, 16)).reshape(1, 512)
    b3rep = jnp.broadcast_to(jnp.pad(b3, (0, 11)).reshape(1, 16),
                             (8, 16)).reshape(1, 128)
    batchp = jnp.concatenate(
        [batch.astype(jnp.int32),
         jnp.full((NP - N,), 99, jnp.int32)])
    onehotB = (batchp.reshape(NF, 1, 8) == jnp.arange(G, dtype=jnp.int32)
               .reshape(1, G, 1)).astype(jnp.float32).reshape(NF, G * 8)
    kmask = ((jnp.arange(G * 8, dtype=jnp.int32) % 8)[:, None] ==
             (jnp.arange(128, dtype=jnp.int32) // 16)[None, :]
             ).astype(jnp.float32)
    fold1 = ((jnp.arange(128, dtype=jnp.int32) % 16)[:, None] ==
             jnp.arange(16, dtype=jnp.int32)[None, :]).astype(jnp.float32)
    fold2 = ((jnp.arange(G * 8, dtype=jnp.int32) // 8)[:, None] ==
             jnp.arange(G, dtype=jnp.int32)[None, :]).astype(jnp.float32)
    zrs = jnp.zeros((STRIPE, 16), jnp.float32)

    deg2 = _deg_kernel(eip)
    dinv_lin = _tc1a(deg2.reshape(NCORES, NF // 16, 128))
    dinvf = jnp.broadcast_to(
        dinv_lin.reshape(NP, 1), (NP, 16)).reshape(NF, 128)
    g1f = _tc1b(dinvf, xf)
    acc1 = _prop1(g1f.reshape(NP, 16), eip, zrs)
    g2fs = _tc2(acc1.reshape(NCORES, NF, 128), g1f, dinvf, w1perm, b1perm)
    acc2 = _prop4(*[g.reshape(NP, 16) for g in g2fs], eip, zrs)
    g3f = _tc3(acc2.reshape(NCORES, 4, NF, 128), g2fs, dinvf,
               w2perm, b2perm, w3perm)
    acc3 = _prop1(g3f.reshape(NP, 16), eip, zrs)
    (pooled,) = _tc4(acc3.reshape(NCORES, NF, 128), g3f, dinvf, b3rep,
                     onehotB, kmask, fold1, fold2)
    return pooled[:, :5]


def kernel(x, edge_index, batch, W1, b1, W2, b2, W3, b3):
    return _run(x, edge_index, batch, W1, b1, W2, b2, W3, b3)


# speedup vs baseline: 53.7659x; 1.3491x over previous
import functools

import jax
import jax.numpy as jnp
from jax import lax
from jax.experimental import pallas as pl
from jax.experimental.pallas import tpu as pltpu
from jax.experimental.pallas import tpu_sc as plsc

N = 100000
E = 1600000
G = 32
NP = 100352
STRIPE = NP // 16
ZROWS = STRIPE // 8
NCORES = 2
NJ = 5
NCHUNK = 80
EPT = NCHUNK * NJ * 128
EPAD = NCORES * 16 * EPT


def _sc_mesh():
    return plsc.VectorSubcoreMesh(core_axis_name="c", subcore_axis_name="s")


@functools.partial(
    pl.kernel,
    out_type=jax.ShapeDtypeStruct((NCORES, NP), jnp.float32),
    mesh=_sc_mesh(),
    compiler_params=pltpu.CompilerParams(use_tc_tiling_on_sc=False),
    scratch_types=[
        pltpu.VMEM((2, NJ, 128), jnp.int32),
        pltpu.VMEM((128,), jnp.float32),
        pltpu.VMEM((ZROWS,), jnp.float32),
        pltpu.VMEM_SHARED((NP,), jnp.float32),
        pltpu.SemaphoreType.DMA,
        pltpu.SemaphoreType.DMA,
    ],
)
def _deg_kernel(ei_hbm, out_hbm, dstv, ones_v, zb, acc, sem0, sem1):
    c = lax.axis_index("c")
    s = lax.axis_index("s")
    sems = (sem0, sem1)

    for i in range(8):
        ones_v[pl.ds(16 * i, 16)] = jnp.ones((16,), jnp.float32)

    def zinit(i, carry):
        zb[pl.ds(16 * i, 16)] = jnp.zeros((16,), jnp.float32)
        return carry

    lax.fori_loop(0, ZROWS // 16, zinit, 0)
    for k in range(8):
        pltpu.sync_copy(zb, acc.at[pl.ds(s * STRIPE + k * ZROWS, ZROWS)])
    plsc.subcore_barrier()

    def idx(p, ci):
        pltpu.sync_copy(ei_hbm.at[1, c, s, ci], dstv.at[p])

    def fire_s(p):
        for j in range(NJ):
            pltpu.async_copy(ones_v, acc.at[dstv.at[p, j]], sems[p],
                             add=True)

    def drain_s(p):
        for j in range(NJ):
            pltpu.make_async_copy(ones_v, acc.at[dstv.at[p, j]],
                                  sems[p]).wait()

    idx(0, 0)

    def pair(i, carry):
        fire_s(0)
        idx(1, 2 * i + 1)
        fire_s(1)
        drain_s(0)
        idx(0, 2 * i + 2)
        drain_s(1)
        return carry

    lax.fori_loop(0, NCHUNK // 2 - 1, pair, 0)
    fire_s(0)
    idx(1, NCHUNK - 1)
    fire_s(1)
    drain_s(0)
    drain_s(1)
    plsc.subcore_barrier()
    pltpu.sync_copy(acc.at[pl.ds(s * STRIPE, STRIPE)],
                    out_hbm.at[c, pl.ds(s * STRIPE, STRIPE)])


def _make_prop(nsl):
    @functools.partial(
        pl.kernel,
        out_type=jax.ShapeDtypeStruct((NCORES, nsl, NP, 16), jnp.float32),
        mesh=_sc_mesh(),
        compiler_params=pltpu.CompilerParams(use_tc_tiling_on_sc=False),
        scratch_types=[
            pltpu.VMEM((2, NJ, 128), jnp.int32),
            pltpu.VMEM((2, NJ, 128), jnp.int32),
            pltpu.VMEM((2, NJ * 128, 16), jnp.float32),
            pltpu.VMEM_SHARED((NP, 16), jnp.float32),
        ] + [pltpu.SemaphoreType.DMA] * 8,
    )
    def _prop(*refs):
        gs = refs[0:nsl]
        ei_hbm = refs[nsl]
        zeros_hbm, out_hbm = refs[nsl + 1], refs[nsl + 2]
        srcv, dstv, rows, acc = refs[nsl + 3:nsl + 7]
        semg = refs[nsl + 7:nsl + 9]
        sems = refs[nsl + 9:nsl + 11]
        semsrc = refs[nsl + 11:nsl + 13]
        semdst = refs[nsl + 13:nsl + 15]
        c = lax.axis_index("c")
        s = lax.axis_index("s")

        for sl in range(nsl):
            g = gs[sl]

            def load_src(p, ci):
                pltpu.async_copy(ei_hbm.at[0, c, s, ci], srcv.at[p],
                                 semsrc[p])

            def wait_src(p):
                pltpu.make_async_copy(ei_hbm.at[0, c, s, 0], srcv.at[p],
                                      semsrc[p]).wait()

            def load_dst(p, ci):
                pltpu.async_copy(ei_hbm.at[1, c, s, ci], dstv.at[p],
                                 semdst[p])

            def wait_dst(p):
                pltpu.make_async_copy(ei_hbm.at[1, c, s, 0], dstv.at[p],
                                      semdst[p]).wait()

            def fire_gv(p):
                for j in range(NJ):
                    pltpu.async_copy(
                        g.at[srcv.at[p, j]],
                        rows.at[p, pl.ds(128 * j, 128), :], semg[p])

            def drain_g(p):
                for j in range(NJ):
                    pltpu.make_async_copy(
                        g.at[srcv.at[p, j]],
                        rows.at[p, pl.ds(128 * j, 128), :], semg[p]).wait()

            def fire_s(p):
                for j in range(NJ):
                    pltpu.async_copy(
                        rows.at[p, pl.ds(128 * j, 128), :],
                        acc.at[dstv.at[p, j]], sems[p], add=True)

            def drain_s(p):
                for j in range(NJ):
                    pltpu.make_async_copy(
                        rows.at[p, pl.ds(128 * j, 128), :],
                        acc.at[dstv.at[p, j]], sems[p]).wait()

            pltpu.sync_copy(zeros_hbm, acc.at[pl.ds(s * STRIPE, STRIPE)])
            plsc.subcore_barrier()

            load_src(0, 0)
            load_dst(0, 0)
            load_src(1, 1)
            load_dst(1, 1)
            wait_src(0)
            fire_gv(0)

            def pair(i, carry):
                a = 2 * i
                wait_src(1)
                fire_gv(1)
                drain_g(0)
                load_src(0, a + 2)
                wait_dst(0)
                fire_s(0)
                drain_g(1)
                load_src(1, a + 3)
                wait_dst(1)
                fire_s(1)
                drain_s(0)
                load_dst(0, a + 2)
                wait_src(0)
                fire_gv(0)
                drain_s(1)
                load_dst(1, a + 3)
                return carry

            lax.fori_loop(0, NCHUNK // 2 - 1, pair, 0)
            wait_src(1)
            fire_gv(1)
            drain_g(0)
            wait_dst(0)
            fire_s(0)
            drain_g(1)
            wait_dst(1)
            fire_s(1)
            drain_s(0)
            drain_s(1)

            plsc.subcore_barrier()
            pltpu.sync_copy(
                acc.at[pl.ds(s * STRIPE, STRIPE)],
                out_hbm.at[c, sl, pl.ds(s * STRIPE, STRIPE)])

    return _prop


_prop1 = _make_prop(1)
_prop4 = _make_prop(4)


_TC_GRID = 16
NF = NP * 16 // 128
_FB = NF // _TC_GRID
_RB = STRIPE


def _frow_spec():
    return pl.BlockSpec((_FB, 128), lambda i: (i, 0))


def _fpair_spec():
    return pl.BlockSpec((NCORES, _FB, 128), lambda i: (0, i, 0))


def _full_spec(r, c):
    return pl.BlockSpec((r, c), lambda i: (0, 0))


_TCP = pltpu.CompilerParams(vmem_limit_bytes=100 * 2**20)


def _tc1a_body(deg_ref, dinv_ref):
    dinv_ref[...] = lax.rsqrt(deg_ref[0] + deg_ref[1] + 1.0)


def _tc1a(deg2f):
    return pl.pallas_call(
        _tc1a_body,
        grid=(1,),
        compiler_params=_TCP,
        in_specs=[pl.BlockSpec((NCORES, NF // 16, 128),
                               lambda i: (0, 0, 0))],
        out_specs=pl.BlockSpec((NF // 16, 128), lambda i: (0, 0)),
        out_shape=jax.ShapeDtypeStruct((NF // 16, 128), jnp.float32),
    )(deg2f)


def _tc1b_body(dinv_ref, xf_ref, g1_ref):
    g1_ref[...] = dinv_ref[...] * xf_ref[...]


def _tc1b(dinvf, xf):
    return pl.pallas_call(
        _tc1b_body,
        grid=(_TC_GRID,),
        compiler_params=_TCP,
        in_specs=[_frow_spec(), _frow_spec()],
        out_specs=_frow_spec(),
        out_shape=jax.ShapeDtypeStruct((NF, 128), jnp.float32),
    )(dinvf, xf)


def _tc2_body(acc1_ref, g1_ref, dinv_ref, w1_ref, b1_ref, *g2_refs):
    dinv = dinv_ref[...]
    p1 = dinv * (acc1_ref[0] + acc1_ref[1] + g1_ref[...])
    mid = jnp.maximum(
        jnp.dot(p1, w1_ref[...], preferred_element_type=jnp.float32,
                precision=lax.Precision.DEFAULT) + b1_ref[...], 0.0)
    for sl in range(4):
        g2_refs[sl][...] = dinv * mid[:, 128 * sl:128 * (sl + 1)]


def _tc2(acc1f, g1f, dinvf, w1perm, b1perm):
    return pl.pallas_call(
        _tc2_body,
        grid=(_TC_GRID,),
        compiler_params=_TCP,
        in_specs=[_fpair_spec(), _frow_spec(), _frow_spec(),
                  _full_spec(128, 512), _full_spec(1, 512)],
        out_specs=[_frow_spec()] * 4,
        out_shape=[jax.ShapeDtypeStruct((NF, 128), jnp.float32)] * 4,
    )(acc1f, g1f, dinvf, w1perm, b1perm)


def _tc3_body(acc2_ref, g2a_ref, g2b_ref, g2c_ref, g2d_ref, dinv_ref,
              w2_ref, b2_ref, w3_ref, g3_ref):
    dinv = dinv_ref[...]
    g2r = (g2a_ref, g2b_ref, g2c_ref, g2d_ref)
    p2 = jnp.concatenate(
        [dinv * (acc2_ref[0, sl] + acc2_ref[1, sl] + g2r[sl][...])
         for sl in range(4)], axis=1)
    mid = jnp.maximum(
        jnp.dot(p2, w2_ref[...], preferred_element_type=jnp.float32,
                precision=lax.Precision.DEFAULT) + b2_ref[...], 0.0)
    t3 = jnp.dot(mid, w3_ref[...], preferred_element_type=jnp.float32,
                 precision=lax.Precision.DEFAULT)
    g3_ref[...] = dinv * t3


def _tc3(acc2f, g2fs, dinvf, w2perm, b2perm, w3perm):
    return pl.pallas_call(
        _tc3_body,
        grid=(_TC_GRID,),
        compiler_params=_TCP,
        in_specs=[pl.BlockSpec((NCORES, 4, _FB, 128), lambda i: (0, 0, i, 0))]
                 + [_frow_spec()] * 4
                 + [_frow_spec(), _full_spec(512, 512), _full_spec(1, 512),
                    _full_spec(512, 128)],
        out_specs=_frow_spec(),
        out_shape=jax.ShapeDtypeStruct((NF, 128), jnp.float32),
    )(acc2f, *g2fs, dinvf, w2perm, b2perm, w3perm)


def _tc4_body(acc3_ref, g3_ref, dinv_ref, b3_ref, oh_ref, km_ref,
              f1_ref, f2_ref, out_ref, sums_ref, cnts_ref):
    i = pl.program_id(0)

    @pl.when(i == 0)
    def _():
        sums_ref[...] = jnp.zeros((G * 8, 128), jnp.float32)
        cnts_ref[...] = jnp.zeros((G * 8, 128), jnp.float32)

    of = dinv_ref[...] * (acc3_ref[0] + acc3_ref[1] + g3_ref[...]) \
        + b3_ref[...]
    oh = oh_ref[...]
    sums_ref[...] += lax.dot_general(
        oh, of, (((0,), (0,)), ((), ())),
        preferred_element_type=jnp.float32,
        precision=lax.Precision.HIGHEST)
    cnts_ref[...] += lax.dot_general(
        oh, jnp.ones((_FB, 128), jnp.float32), (((0,), (0,)), ((), ())),
        preferred_element_type=jnp.float32,
        precision=lax.Precision.HIGHEST)

    @pl.when(i == _TC_GRID - 1)
    def _():
        km = km_ref[...]
        f1 = f1_ref[...]
        f2 = f2_ref[...]
        sE = lax.dot_general(
            f2, jnp.dot(sums_ref[...] * km, f1,
                        preferred_element_type=jnp.float32,
                        precision=lax.Precision.HIGHEST),
            (((0,), (0,)), ((), ())),
            preferred_element_type=jnp.float32,
            precision=lax.Precision.HIGHEST)
        cE = lax.dot_general(
            f2, jnp.dot(cnts_ref[...] * km, f1,
                        preferred_element_type=jnp.float32,
                        precision=lax.Precision.HIGHEST),
            (((0,), (0,)), ((), ())),
            preferred_element_type=jnp.float32,
            precision=lax.Precision.HIGHEST)
        out_ref[...] = sE / jnp.maximum(cE, 1.0)


def _tc4(acc3f, g3f, dinvf, b3rep, onehotB, kmask, fold1, fold2):
    return pl.pallas_call(
        _tc4_body,
        grid=(_TC_GRID,),
        compiler_params=_TCP,
        in_specs=[_fpair_spec(), _frow_spec(), _frow_spec(),
                  _full_spec(1, 128),
                  pl.BlockSpec((_FB, G * 8), lambda i: (i, 0)),
                  _full_spec(G * 8, 128), _full_spec(128, 16),
                  _full_spec(G * 8, G)],
        out_specs=[_full_spec(G, 16)],
        out_shape=[jax.ShapeDtypeStruct((G, 16), jnp.float32)],
        scratch_shapes=[pltpu.VMEM((G * 8, 128), jnp.float32),
                        pltpu.VMEM((G * 8, 128), jnp.float32)],
    )(acc3f, g3f, dinvf, b3rep, onehotB, kmask, fold1, fold2)


@jax.jit
def _run(x, edge_index, batch, W1, b1, W2, b2, W3, b3):
    pad = N + (jnp.arange(EPAD - E, dtype=jnp.int32) % (NP - N))
    eip = jnp.concatenate(
        [edge_index.astype(jnp.int32),
         jnp.broadcast_to(pad[None, :], (2, EPAD - E))],
        axis=1).reshape(2, NCORES, 16, NCHUNK, NJ, 128)

    xf = jnp.pad(x, ((0, NP - N), (0, 13))).reshape(NF, 128)
    eye8 = jnp.eye(8, dtype=jnp.float32)
    w1p = jnp.pad(W1, ((0, 13), (0, 0)))
    w1perm = jnp.einsum('kK,ism->kisKm', eye8,
                        w1p.reshape(16, 4, 16)).reshape(128, 512)
    w2perm = jnp.einsum('kK,amAM->akmAKM', eye8,
                        W2.reshape(4, 16, 4, 16)).reshape(512, 512)
    w3p = jnp.pad(W3, ((0, 0), (0, 11)))
    w3perm = jnp.einsum('kK,amM->akmKM', eye8,
                        w3p.reshape(4, 16, 16)).reshape(512, 128)
    b1perm = jnp.broadcast_to(b1.reshape(4, 1, 16), (4, 8, 16)).reshape(1, 512)
    b2perm = jnp.broadcast_to(b2.reshape(4, 1, 16), (4, 8, 16)).reshape(1, 512)
    b3rep = jnp.broadcast_to(jnp.pad(b3, (0, 11)).reshape(1, 16),
                             (8, 16)).reshape(1, 128)
    batchp = jnp.concatenate(
        [batch.astype(jnp.int32),
         jnp.full((NP - N,), 99, jnp.int32)])
    onehotB = (batchp.reshape(NF, 1, 8) == jnp.arange(G, dtype=jnp.int32)
               .reshape(1, G, 1)).astype(jnp.float32).reshape(NF, G * 8)
    kmask = ((jnp.arange(G * 8, dtype=jnp.int32) % 8)[:, None] ==
             (jnp.arange(128, dtype=jnp.int32) // 16)[None, :]
             ).astype(jnp.float32)
    fold1 = ((jnp.arange(128, dtype=jnp.int32) % 16)[:, None] ==
             jnp.arange(16, dtype=jnp.int32)[None, :]).astype(jnp.float32)
    fold2 = ((jnp.arange(G * 8, dtype=jnp.int32) // 8)[:, None] ==
             jnp.arange(G, dtype=jnp.int32)[None, :]).astype(jnp.float32)
    zrs = jnp.zeros((STRIPE, 16), jnp.float32)

    deg2 = _deg_kernel(eip)
    dinv_lin = _tc1a(deg2.reshape(NCORES, NF // 16, 128))
    dinvf = jnp.broadcast_to(
        dinv_lin.reshape(NP, 1), (NP, 16)).reshape(NF, 128)
    g1f = _tc1b(dinvf, xf)
    acc1 = _prop1(g1f.reshape(NP, 16), eip, zrs)
    g2fs = _tc2(acc1.reshape(NCORES, NF, 128), g1f, dinvf, w1perm, b1perm)
    acc2 = _prop4(*[g.reshape(NP, 16) for g in g2fs], eip, zrs)
    g3f = _tc3(acc2.reshape(NCORES, 4, NF, 128), g2fs, dinvf,
               w2perm, b2perm, w3perm)
    acc3 = _prop1(g3f.reshape(NP, 16), eip, zrs)
    (pooled,) = _tc4(acc3.reshape(NCORES, NF, 128), g3f, dinvf, b3rep,
                     onehotB, kmask, fold1, fold2)
    return pooled[:, :5]


def kernel(x, edge_index, batch, W1, b1, W2, b2, W3, b3):
    return _run(x, edge_index, batch, W1, b1, W2, b2, W3, b3)
